# baseline (speedup 1.0000x reference)
import functools

import jax
import jax.numpy as jnp
from jax import lax
from jax.experimental import pallas as pl
from jax.experimental.pallas import tpu as pltpu
from jax.experimental.pallas import tpu_sc as plsc

N = 10000
E = 320000
D = 128
NC = 2
NS = 16
NW = NC * NS
EPW = E // NW
CHUNK = 80
NCHUNK = EPW // CHUNK
ZROWS = 624
REM_BASE = NS * ZROWS
REM = N - REM_BASE
ZBUF = 16

NBUF_G = 5
GGRP = NBUF_G * CHUNK
NGRP_G = EPW // GGRP

NBUF_M = 2
MGRP = NBUF_M * CHUNK
NGRP_M = NCHUNK // NBUF_M

_MESH = plsc.VectorSubcoreMesh(
    core_axis_name="c", subcore_axis_name="s", num_cores=NC, num_subcores=NS)


def _relu_add_rows(g_ref, e_ref):
    def row(r, carry):
        for k in range(D // 16):
            sl = pl.ds(k * 16, 16)
            g_ref[r, sl] = jnp.maximum(g_ref[r, sl] + e_ref[r, sl], 0.0)
        return carry
    lax.fori_loop(0, CHUNK, row, 0, unroll=2)


_MSG_SCRATCH = (
    [pltpu.VMEM_SHARED((N, D), jnp.float32)]
    + [pltpu.VMEM((MGRP,), jnp.int32)]
    + [pltpu.VMEM((CHUNK,), jnp.int32) for _ in range(NBUF_M)]
    + [pltpu.VMEM((CHUNK, D), jnp.float32) for _ in range(NBUF_M)]
    + [pltpu.VMEM((CHUNK, D), jnp.float32) for _ in range(NBUF_M)]
    + [pltpu.VMEM((ZBUF, D), jnp.float32)]
    + [pltpu.SemaphoreType.DMA for _ in range(4 * NBUF_M)]
)


@functools.partial(
    pl.kernel,
    out_type=jax.ShapeDtypeStruct((NC, N, D), jnp.float32),
    mesh=_MESH,
    scratch_types=_MSG_SCRATCH,
)
def _sc_message(nf_hbm, ef_hbm, src_hbm, dst_hbm, out_hbm, *sc):
    acc = sc[0]
    gd = sc[1]
    ss = sc[2:2 + NBUF_M]
    gb = sc[4:4 + NBUF_M]
    eb = sc[6:6 + NBUF_M]
    zbuf = sc[8]
    isem = sc[9:9 + NBUF_M]
    gsem = sc[11:11 + NBUF_M]
    esem = sc[13:13 + NBUF_M]
    ssem = sc[15:15 + NBUF_M]

    c = lax.axis_index("c")
    s = lax.axis_index("s")
    wid = s * NC + c
    base_w = wid * EPW

    def zrow(r, carry):
        for k in range(D // 16):
            zbuf[r, pl.ds(k * 16, 16)] = jnp.zeros((16,), jnp.float32)
        return carry
    lax.fori_loop(0, ZBUF, zrow, 0)
    for j in range(ZROWS // ZBUF):
        pltpu.sync_copy(zbuf, acc.at[pl.ds(s * ZROWS + j * ZBUF, ZBUF)])

    @pl.when(s == NS - 1)
    def _():
        pltpu.sync_copy(zbuf.at[pl.ds(0, REM)], acc.at[pl.ds(REM_BASE, REM)])
    plsc.subcore_barrier()

    def mgroup(base_g, first):
        pltpu.sync_copy(dst_hbm.at[pl.ds(base_g, MGRP)], gd)
        hs = []
        for b in range(NBUF_M):
            base = base_g + b * CHUNK
            hi = pltpu.async_copy(src_hbm.at[pl.ds(base, CHUNK)], ss[b], isem[b])
            hg = pltpu.async_copy(nf_hbm.at[gd.at[pl.ds(b * CHUNK, CHUNK)]],
                                  gb[b], gsem[b])
            he = pltpu.async_copy(ef_hbm.at[pl.ds(base, CHUNK)], eb[b], esem[b])
            hs.append((hi, hg, he))
        for b in range(NBUF_M):
            hi, hg, he = hs[b]
            hg.wait()
            he.wait()
            _relu_add_rows(gb[b], eb[b])
            hi.wait()
            pltpu.sync_copy(gb[b], acc.at[ss[b]], add=True)

    mgroup(base_w, True)
    lax.fori_loop(1, NGRP_M,
                  lambda g, carry: (mgroup(base_w + g * MGRP, False), carry)[1],
                  0)

    tbase = base_w + NGRP_M * MGRP
    pltpu.sync_copy(dst_hbm.at[pl.ds(tbase, CHUNK)], gd.at[pl.ds(0, CHUNK)])
    hi = pltpu.async_copy(src_hbm.at[pl.ds(tbase, CHUNK)], ss[0], isem[0])
    hg = pltpu.async_copy(nf_hbm.at[gd.at[pl.ds(0, CHUNK)]], gb[0], gsem[0])
    he = pltpu.async_copy(ef_hbm.at[pl.ds(tbase, CHUNK)], eb[0], esem[0])
    hg.wait()
    he.wait()
    _relu_add_rows(gb[0], eb[0])
    hi.wait()
    pltpu.sync_copy(gb[0], acc.at[ss[0]], add=True)

    plsc.subcore_barrier()
    sl = pl.ds(s * ZROWS, ZROWS)
    pltpu.sync_copy(acc.at[sl], out_hbm.at[c, sl])

    @pl.when(s == NS - 1)
    def _():
        rsl = pl.ds(REM_BASE, REM)
        pltpu.sync_copy(acc.at[rsl], out_hbm.at[c, rsl])


_GATHER_SCRATCH = (
    [pltpu.VMEM((GGRP,), jnp.int32) for _ in range(2)]
    + [pltpu.VMEM((CHUNK, D), jnp.float32) for _ in range(2 * NBUF_G)]
    + [pltpu.SemaphoreType.DMA for _ in range(4 * NBUF_G)]
)


@functools.partial(
    pl.kernel,
    out_type=(
        jax.ShapeDtypeStruct((E, D), jnp.float32),
        jax.ShapeDtypeStruct((E, D), jnp.float32),
    ),
    mesh=_MESH,
    scratch_types=_GATHER_SCRATCH,
)
def _sc_edge_gather(nf_hbm, src_hbm, dst_hbm, gi_hbm, gj_hbm, *sc):
    isrc = sc[0]
    idst = sc[1]
    gbi = sc[2:2 + NBUF_G]
    gbj = sc[7:7 + NBUF_G]
    gsi = sc[12:12 + NBUF_G]
    gsj = sc[17:17 + NBUF_G]
    wsi = sc[22:22 + NBUF_G]
    wsj = sc[27:27 + NBUF_G]

    c = lax.axis_index("c")
    s = lax.axis_index("s")
    wid = s * NC + c
    base_w = wid * EPW

    def group(base_g, first):
        pltpu.sync_copy(src_hbm.at[pl.ds(base_g, GGRP)], isrc)
        pltpu.sync_copy(dst_hbm.at[pl.ds(base_g, GGRP)], idst)
        hs = []
        for b in range(NBUF_G):
            if not first:
                pltpu.make_async_copy(gbi[b], gi_hbm.at[pl.ds(0, CHUNK)],
                                      wsi[b]).wait()
                pltpu.make_async_copy(gbj[b], gj_hbm.at[pl.ds(0, CHUNK)],
                                      wsj[b]).wait()
            h1 = pltpu.async_copy(nf_hbm.at[isrc.at[pl.ds(b * CHUNK, CHUNK)]],
                                  gbi[b], gsi[b])
            h2 = pltpu.async_copy(nf_hbm.at[idst.at[pl.ds(b * CHUNK, CHUNK)]],
                                  gbj[b], gsj[b])
            hs.append((h1, h2))
        for b in range(NBUF_G):
            h1, h2 = hs[b]
            sl = pl.ds(base_g + b * CHUNK, CHUNK)
            h1.wait()
            pltpu.async_copy(gbi[b], gi_hbm.at[sl], wsi[b])
            h2.wait()
            pltpu.async_copy(gbj[b], gj_hbm.at[sl], wsj[b])

    group(base_w, True)
    lax.fori_loop(1, NGRP_G,
                  lambda g, carry: (group(base_w + g * GGRP, False), carry)[1],
                  0)
    for b in range(NBUF_G):
        pltpu.make_async_copy(gbi[b], gi_hbm.at[pl.ds(0, CHUNK)], wsi[b]).wait()
        pltpu.make_async_copy(gbj[b], gj_hbm.at[pl.ds(0, CHUNK)], wsj[b]).wait()


def _ln(x, g, b):
    m = jnp.mean(x, axis=-1, keepdims=True)
    xc = x - m
    v = jnp.mean(xc * xc, axis=-1, keepdims=True)
    return xc * jax.lax.rsqrt(v + 1e-5) * g + b


BN = 1000


def _node_mlp_body(nf, p0, p1, eps, w1, b1, lg1, lb1, w2, b2, bng, bnb,
                   out_ln, out_relu):
    h = (1.0 + eps[0, 0]) * nf[...] + p0[...] + p1[...]
    t = jnp.dot(h, w1[...], preferred_element_type=jnp.float32) + b1[...]
    t = _ln(t, lg1[...], lb1[...])
    t = jnp.maximum(t, 0.0)
    u = jnp.dot(t, w2[...], preferred_element_type=jnp.float32) + b2[...]
    y = _ln(u, bng[...], bnb[...])
    out_ln[...] = y
    out_relu[...] = jnp.maximum(y, 0.0)


def _tc_node_mlp(nf, p0, p1, eps, w1, b1, lg1, lb1, w2, b2, bng, bnb):
    grid = (N // BN,)
    row_spec = pl.BlockSpec((BN, D), lambda i: (i, 0))
    full = lambda shape: pl.BlockSpec(shape, lambda i: (0, 0))
    return pl.pallas_call(
        _node_mlp_body,
        grid=grid,
        in_specs=[
            row_spec, row_spec, row_spec,
            full((1, 1)),
            full((D, 2 * D)), full((1, 2 * D)), full((1, 2 * D)), full((1, 2 * D)),
            full((2 * D, D)), full((1, D)), full((1, D)), full((1, D)),
        ],
        out_specs=[row_spec, row_spec],
        out_shape=[
            jax.ShapeDtypeStruct((N, D), jnp.float32),
            jax.ShapeDtypeStruct((N, D), jnp.float32),
        ],
    )(nf, p0, p1, eps, w1, b1, lg1, lb1, w2, b2, bng, bnb)


BM = 512


def _edge_mlp_body(gi, gj, ef, wa, wb, wc, b1, lg, lb, w2, b2, out):
    bf = jnp.bfloat16
    efv = ef[...]
    s = (gi[...] + gj[...]).astype(bf)
    a = jnp.abs(gi[...] - gj[...]).astype(bf)
    t = jnp.dot(s, wa[...], preferred_element_type=jnp.float32)
    t += jnp.dot(a, wb[...], preferred_element_type=jnp.float32)
    t += jnp.dot(efv.astype(bf), wc[...], preferred_element_type=jnp.float32)
    t += b1[...]
    t = _ln(t, lg[...], lb[...])
    t = jnp.maximum(t, 0.0).astype(bf)
    u = jnp.dot(t, w2[...], preferred_element_type=jnp.float32) + b2[...]
    out[...] = u + efv


def _tc_edge_mlp(gi, gj, ef, wa, wb, wc, b1, lg, lb, w2, b2):
    grid = (E // BM,)
    full = lambda shape: pl.BlockSpec(shape, lambda i: (0, 0))
    row_spec = pl.BlockSpec((BM, D), lambda i: (i, 0))
    return pl.pallas_call(
        _edge_mlp_body,
        grid=grid,
        in_specs=[
            row_spec, row_spec, row_spec,
            full((D, 3 * D)), full((D, 3 * D)), full((D, 3 * D)),
            full((1, 3 * D)), full((1, 3 * D)), full((1, 3 * D)),
            full((3 * D, D)), full((1, D)),
        ],
        out_specs=row_spec,
        out_shape=jax.ShapeDtypeStruct((E, D), jnp.float32),
    )(gi, gj, ef,
      wa.astype(jnp.bfloat16), wb.astype(jnp.bfloat16), wc.astype(jnp.bfloat16),
      b1, lg, lb, w2.astype(jnp.bfloat16), b2)


def kernel(node_feats, edge_feats, edge_index, params):
    src = edge_index[0]
    dst = edge_index[1]
    nf = node_feats
    ef = edge_feats
    nf_ln = node_feats
    num_layers = len(params)
    for l in range(num_layers):
        p = params["layer%d" % l]
        parts = _sc_message(nf, ef, src, dst)
        nf_ln, nf_relu = _tc_node_mlp(
            nf, parts[0], parts[1],
            p["eps"].reshape(1, 1),
            p["cW1"], p["cb1"].reshape(1, -1),
            p["cln_g"].reshape(1, -1), p["cln_b"].reshape(1, -1),
            p["cW2"], p["cb2"].reshape(1, -1),
            p["bn_g"].reshape(1, -1), p["bn_b"].reshape(1, -1),
        )
        gi, gj = _sc_edge_gather(nf_ln, src, dst)
        ef = _tc_edge_mlp(
            gi, gj, ef,
            p["eW1"][:D], p["eW1"][D:2 * D], p["eW1"][2 * D:],
            p["eb1"].reshape(1, -1),
            p["eln_g"].reshape(1, -1), p["eln_b"].reshape(1, -1),
            p["eW2"], p["eb2"].reshape(1, -1),
        )
        nf = nf_relu
    return nf_ln, ef


# speedup vs baseline: 1.0327x; 1.0327x over previous
import functools

import jax
import jax.numpy as jnp
from jax import lax
from jax.experimental import pallas as pl
from jax.experimental.pallas import tpu as pltpu
from jax.experimental.pallas import tpu_sc as plsc

N = 10000
E = 320000
D = 128
NC = 2
NS = 16
NW = NC * NS
EPW = E // NW
CHUNK = 80
NCHUNK = EPW // CHUNK
ZROWS = 624
REM_BASE = NS * ZROWS
REM = N - REM_BASE
ZBUF = 16

NBUF_G = 5
GGRP = NBUF_G * CHUNK
NGRP_G = EPW // GGRP

NBUF_M = 2
MGRP = NBUF_M * CHUNK
NGRP_M = NCHUNK // NBUF_M

_MESH = plsc.VectorSubcoreMesh(
    core_axis_name="c", subcore_axis_name="s", num_cores=NC, num_subcores=NS)


def _relu_add_rows(g_ref, e_ref):
    def row(r, carry):
        for k in range(D // 16):
            sl = pl.ds(k * 16, 16)
            g_ref[r, sl] = jnp.maximum(g_ref[r, sl] + e_ref[r, sl], 0.0)
        return carry
    lax.fori_loop(0, CHUNK, row, 0, unroll=2)


_MSG_SCRATCH = (
    [pltpu.VMEM_SHARED((N, D), jnp.float32)]
    + [pltpu.VMEM((MGRP,), jnp.int32)]
    + [pltpu.VMEM((CHUNK,), jnp.int32) for _ in range(NBUF_M)]
    + [pltpu.VMEM((CHUNK, D), jnp.float32) for _ in range(NBUF_M)]
    + [pltpu.VMEM((CHUNK, D), jnp.float32) for _ in range(NBUF_M)]
    + [pltpu.VMEM((ZBUF, D), jnp.float32)]
    + [pltpu.SemaphoreType.DMA for _ in range(4 * NBUF_M)]
)


@functools.partial(
    pl.kernel,
    out_type=jax.ShapeDtypeStruct((NC, N, D), jnp.float32),
    mesh=_MESH,
    scratch_types=_MSG_SCRATCH,
)
def _sc_message(nf_hbm, ef_hbm, src_hbm, dst_hbm, out_hbm, *sc):
    acc = sc[0]
    gd = sc[1]
    ss = sc[2:2 + NBUF_M]
    gb = sc[4:4 + NBUF_M]
    eb = sc[6:6 + NBUF_M]
    zbuf = sc[8]
    isem = sc[9:9 + NBUF_M]
    gsem = sc[11:11 + NBUF_M]
    esem = sc[13:13 + NBUF_M]
    ssem = sc[15:15 + NBUF_M]

    c = lax.axis_index("c")
    s = lax.axis_index("s")
    wid = s * NC + c
    base_w = wid * EPW

    def zrow(r, carry):
        for k in range(D // 16):
            zbuf[r, pl.ds(k * 16, 16)] = jnp.zeros((16,), jnp.float32)
        return carry
    lax.fori_loop(0, ZBUF, zrow, 0)
    for j in range(ZROWS // ZBUF):
        pltpu.sync_copy(zbuf, acc.at[pl.ds(s * ZROWS + j * ZBUF, ZBUF)])

    @pl.when(s == NS - 1)
    def _():
        pltpu.sync_copy(zbuf.at[pl.ds(0, REM)], acc.at[pl.ds(REM_BASE, REM)])
    plsc.subcore_barrier()

    def mgroup(base_g, first):
        pltpu.sync_copy(dst_hbm.at[pl.ds(base_g, MGRP)], gd)
        hs = []
        for b in range(NBUF_M):
            base = base_g + b * CHUNK
            if not first:
                pltpu.make_async_copy(gb[b], acc.at[pl.ds(0, CHUNK)],
                                      ssem[b]).wait()
            hi = pltpu.async_copy(src_hbm.at[pl.ds(base, CHUNK)], ss[b], isem[b])
            hg = pltpu.async_copy(nf_hbm.at[gd.at[pl.ds(b * CHUNK, CHUNK)]],
                                  gb[b], gsem[b])
            he = pltpu.async_copy(ef_hbm.at[pl.ds(base, CHUNK)], eb[b], esem[b])
            hs.append((hi, hg, he))
        for b in range(NBUF_M):
            hi, hg, he = hs[b]
            hg.wait()
            he.wait()
            _relu_add_rows(gb[b], eb[b])
            hi.wait()
            pltpu.async_copy(gb[b], acc.at[ss[b]], ssem[b], add=True)

    mgroup(base_w, True)
    lax.fori_loop(1, NGRP_M,
                  lambda g, carry: (mgroup(base_w + g * MGRP, False), carry)[1],
                  0)

    tbase = base_w + NGRP_M * MGRP
    pltpu.make_async_copy(gb[0], acc.at[pl.ds(0, CHUNK)], ssem[0]).wait()
    pltpu.sync_copy(dst_hbm.at[pl.ds(tbase, CHUNK)], gd.at[pl.ds(0, CHUNK)])
    hi = pltpu.async_copy(src_hbm.at[pl.ds(tbase, CHUNK)], ss[0], isem[0])
    hg = pltpu.async_copy(nf_hbm.at[gd.at[pl.ds(0, CHUNK)]], gb[0], gsem[0])
    he = pltpu.async_copy(ef_hbm.at[pl.ds(tbase, CHUNK)], eb[0], esem[0])
    hg.wait()
    he.wait()
    _relu_add_rows(gb[0], eb[0])
    hi.wait()
    pltpu.async_copy(gb[0], acc.at[ss[0]], ssem[0], add=True)
    for b in range(NBUF_M):
        pltpu.make_async_copy(gb[b], acc.at[pl.ds(0, CHUNK)], ssem[b]).wait()

    plsc.subcore_barrier()
    sl = pl.ds(s * ZROWS, ZROWS)
    pltpu.sync_copy(acc.at[sl], out_hbm.at[c, sl])

    @pl.when(s == NS - 1)
    def _():
        rsl = pl.ds(REM_BASE, REM)
        pltpu.sync_copy(acc.at[rsl], out_hbm.at[c, rsl])


_GATHER_SCRATCH = (
    [pltpu.VMEM((GGRP,), jnp.int32) for _ in range(2)]
    + [pltpu.VMEM((CHUNK, D), jnp.float32) for _ in range(2 * NBUF_G)]
    + [pltpu.SemaphoreType.DMA for _ in range(4 * NBUF_G)]
)


@functools.partial(
    pl.kernel,
    out_type=(
        jax.ShapeDtypeStruct((E, D), jnp.float32),
        jax.ShapeDtypeStruct((E, D), jnp.float32),
    ),
    mesh=_MESH,
    scratch_types=_GATHER_SCRATCH,
)
def _sc_edge_gather(nf_hbm, src_hbm, dst_hbm, gi_hbm, gj_hbm, *sc):
    isrc = sc[0]
    idst = sc[1]
    gbi = sc[2:2 + NBUF_G]
    gbj = sc[7:7 + NBUF_G]
    gsi = sc[12:12 + NBUF_G]
    gsj = sc[17:17 + NBUF_G]
    wsi = sc[22:22 + NBUF_G]
    wsj = sc[27:27 + NBUF_G]

    c = lax.axis_index("c")
    s = lax.axis_index("s")
    wid = s * NC + c
    base_w = wid * EPW

    def group(base_g, first):
        pltpu.sync_copy(src_hbm.at[pl.ds(base_g, GGRP)], isrc)
        pltpu.sync_copy(dst_hbm.at[pl.ds(base_g, GGRP)], idst)
        hs = []
        for b in range(NBUF_G):
            if not first:
                pltpu.make_async_copy(gbi[b], gi_hbm.at[pl.ds(0, CHUNK)],
                                      wsi[b]).wait()
                pltpu.make_async_copy(gbj[b], gj_hbm.at[pl.ds(0, CHUNK)],
                                      wsj[b]).wait()
            h1 = pltpu.async_copy(nf_hbm.at[isrc.at[pl.ds(b * CHUNK, CHUNK)]],
                                  gbi[b], gsi[b])
            h2 = pltpu.async_copy(nf_hbm.at[idst.at[pl.ds(b * CHUNK, CHUNK)]],
                                  gbj[b], gsj[b])
            hs.append((h1, h2))
        for b in range(NBUF_G):
            h1, h2 = hs[b]
            sl = pl.ds(base_g + b * CHUNK, CHUNK)
            h1.wait()
            pltpu.async_copy(gbi[b], gi_hbm.at[sl], wsi[b])
            h2.wait()
            pltpu.async_copy(gbj[b], gj_hbm.at[sl], wsj[b])

    group(base_w, True)
    lax.fori_loop(1, NGRP_G,
                  lambda g, carry: (group(base_w + g * GGRP, False), carry)[1],
                  0)
    for b in range(NBUF_G):
        pltpu.make_async_copy(gbi[b], gi_hbm.at[pl.ds(0, CHUNK)], wsi[b]).wait()
        pltpu.make_async_copy(gbj[b], gj_hbm.at[pl.ds(0, CHUNK)], wsj[b]).wait()


def _ln(x, g, b):
    m = jnp.mean(x, axis=-1, keepdims=True)
    xc = x - m
    v = jnp.mean(xc * xc, axis=-1, keepdims=True)
    return xc * jax.lax.rsqrt(v + 1e-5) * g + b


BN = 1000


def _node_mlp_body(nf, p0, p1, eps, w1, b1, lg1, lb1, w2, b2, bng, bnb,
                   out_ln, out_relu):
    h = (1.0 + eps[0, 0]) * nf[...] + p0[...] + p1[...]
    t = jnp.dot(h, w1[...], preferred_element_type=jnp.float32) + b1[...]
    t = _ln(t, lg1[...], lb1[...])
    t = jnp.maximum(t, 0.0)
    u = jnp.dot(t, w2[...], preferred_element_type=jnp.float32) + b2[...]
    y = _ln(u, bng[...], bnb[...])
    out_ln[...] = y
    out_relu[...] = jnp.maximum(y, 0.0)


def _tc_node_mlp(nf, p0, p1, eps, w1, b1, lg1, lb1, w2, b2, bng, bnb):
    grid = (N // BN,)
    row_spec = pl.BlockSpec((BN, D), lambda i: (i, 0))
    full = lambda shape: pl.BlockSpec(shape, lambda i: (0, 0))
    return pl.pallas_call(
        _node_mlp_body,
        grid=grid,
        in_specs=[
            row_spec, row_spec, row_spec,
            full((1, 1)),
            full((D, 2 * D)), full((1, 2 * D)), full((1, 2 * D)), full((1, 2 * D)),
            full((2 * D, D)), full((1, D)), full((1, D)), full((1, D)),
        ],
        out_specs=[row_spec, row_spec],
        out_shape=[
            jax.ShapeDtypeStruct((N, D), jnp.float32),
            jax.ShapeDtypeStruct((N, D), jnp.float32),
        ],
    )(nf, p0, p1, eps, w1, b1, lg1, lb1, w2, b2, bng, bnb)


BM = 512


def _edge_mlp_body(gi, gj, ef, wa, wb, wc, b1, lg, lb, w2, b2, out):
    bf = jnp.bfloat16
    efv = ef[...]
    s = (gi[...] + gj[...]).astype(bf)
    a = jnp.abs(gi[...] - gj[...]).astype(bf)
    t = jnp.dot(s, wa[...], preferred_element_type=jnp.float32)
    t += jnp.dot(a, wb[...], preferred_element_type=jnp.float32)
    t += jnp.dot(efv.astype(bf), wc[...], preferred_element_type=jnp.float32)
    t += b1[...]
    t = _ln(t, lg[...], lb[...])
    t = jnp.maximum(t, 0.0).astype(bf)
    u = jnp.dot(t, w2[...], preferred_element_type=jnp.float32) + b2[...]
    out[...] = u + efv


def _tc_edge_mlp(gi, gj, ef, wa, wb, wc, b1, lg, lb, w2, b2):
    grid = (E // BM,)
    full = lambda shape: pl.BlockSpec(shape, lambda i: (0, 0))
    row_spec = pl.BlockSpec((BM, D), lambda i: (i, 0))
    return pl.pallas_call(
        _edge_mlp_body,
        grid=grid,
        in_specs=[
            row_spec, row_spec, row_spec,
            full((D, 3 * D)), full((D, 3 * D)), full((D, 3 * D)),
            full((1, 3 * D)), full((1, 3 * D)), full((1, 3 * D)),
            full((3 * D, D)), full((1, D)),
        ],
        out_specs=row_spec,
        out_shape=jax.ShapeDtypeStruct((E, D), jnp.float32),
    )(gi, gj, ef,
      wa.astype(jnp.bfloat16), wb.astype(jnp.bfloat16), wc.astype(jnp.bfloat16),
      b1, lg, lb, w2.astype(jnp.bfloat16), b2)


def kernel(node_feats, edge_feats, edge_index, params):
    src = edge_index[0]
    dst = edge_index[1]
    nf = node_feats
    ef = edge_feats
    nf_ln = node_feats
    num_layers = len(params)
    for l in range(num_layers):
        p = params["layer%d" % l]
        parts = _sc_message(nf, ef, src, dst)
        nf_ln, nf_relu = _tc_node_mlp(
            nf, parts[0], parts[1],
            p["eps"].reshape(1, 1),
            p["cW1"], p["cb1"].reshape(1, -1),
            p["cln_g"].reshape(1, -1), p["cln_b"].reshape(1, -1),
            p["cW2"], p["cb2"].reshape(1, -1),
            p["bn_g"].reshape(1, -1), p["bn_b"].reshape(1, -1),
        )
        gi, gj = _sc_edge_gather(nf_ln, src, dst)
        ef = _tc_edge_mlp(
            gi, gj, ef,
            p["eW1"][:D], p["eW1"][D:2 * D], p["eW1"][2 * D:],
            p["eb1"].reshape(1, -1),
            p["eln_g"].reshape(1, -1), p["eln_b"].reshape(1, -1),
            p["eW2"], p["eb2"].reshape(1, -1),
        )
        nf = nf_relu
    return nf_ln, ef


# speedup vs baseline: 1.2677x; 1.2276x over previous
import functools

import jax
import jax.numpy as jnp
from jax import lax
from jax.experimental import pallas as pl
from jax.experimental.pallas import tpu as pltpu
from jax.experimental.pallas import tpu_sc as plsc

N = 10000
E = 320000
D = 128
NC = 2
NS = 16
NW = NC * NS
EPW = E // NW
CHUNK = 80
NCHUNK = EPW // CHUNK
ZROWS = 624
REM_BASE = NS * ZROWS
REM = N - REM_BASE
ZBUF = 16

NBUF_G = 5
GGRP = NBUF_G * CHUNK
NGRP_G = EPW // GGRP

CHUNK_M = 40
NBUF_M = 4
NCHUNK_M = EPW // CHUNK_M
MGRP = NBUF_M * CHUNK_M
NGRP_M = NCHUNK_M // NBUF_M

_MESH = plsc.VectorSubcoreMesh(
    core_axis_name="c", subcore_axis_name="s", num_cores=NC, num_subcores=NS)


def _relu_add_rows(g_ref, e_ref):
    def row(r, carry):
        for k in range(D // 16):
            sl = pl.ds(k * 16, 16)
            g_ref[r, sl] = jnp.maximum(g_ref[r, sl] + e_ref[r, sl], 0.0)
        return carry
    lax.fori_loop(0, CHUNK_M, row, 0, unroll=2)


_MSG_SCRATCH = (
    [pltpu.VMEM_SHARED((N, D), jnp.float32)]
    + [pltpu.VMEM((MGRP,), jnp.int32)]
    + [pltpu.VMEM((CHUNK_M,), jnp.int32) for _ in range(NBUF_M)]
    + [pltpu.VMEM((CHUNK_M, D), jnp.float32) for _ in range(NBUF_M)]
    + [pltpu.VMEM((CHUNK_M, D), jnp.float32) for _ in range(NBUF_M)]
    + [pltpu.VMEM((ZBUF, D), jnp.float32)]
    + [pltpu.SemaphoreType.DMA for _ in range(4 * NBUF_M)]
)


@functools.partial(
    pl.kernel,
    out_type=jax.ShapeDtypeStruct((NC, N, D), jnp.float32),
    mesh=_MESH,
    scratch_types=_MSG_SCRATCH,
)
def _sc_message(nf_hbm, ef_hbm, src_hbm, dst_hbm, out_hbm, *sc):
    acc = sc[0]
    gd = sc[1]
    ss = sc[2:2 + NBUF_M]
    gb = sc[2 + NBUF_M:2 + 2 * NBUF_M]
    eb = sc[2 + 2 * NBUF_M:2 + 3 * NBUF_M]
    zbuf = sc[2 + 3 * NBUF_M]
    sems = sc[3 + 3 * NBUF_M:]
    isem = sems[0:NBUF_M]
    gsem = sems[NBUF_M:2 * NBUF_M]
    esem = sems[2 * NBUF_M:3 * NBUF_M]
    ssem = sems[3 * NBUF_M:4 * NBUF_M]

    c = lax.axis_index("c")
    s = lax.axis_index("s")
    wid = s * NC + c
    base_w = wid * EPW

    def zrow(r, carry):
        for k in range(D // 16):
            zbuf[r, pl.ds(k * 16, 16)] = jnp.zeros((16,), jnp.float32)
        return carry
    lax.fori_loop(0, ZBUF, zrow, 0)
    for j in range(ZROWS // ZBUF):
        pltpu.sync_copy(zbuf, acc.at[pl.ds(s * ZROWS + j * ZBUF, ZBUF)])

    @pl.when(s == NS - 1)
    def _():
        pltpu.sync_copy(zbuf.at[pl.ds(0, REM)], acc.at[pl.ds(REM_BASE, REM)])
    plsc.subcore_barrier()

    def mgroup(base_g, first, nslots=NBUF_M):
        pltpu.sync_copy(dst_hbm.at[pl.ds(base_g, nslots * CHUNK_M)],
                        gd.at[pl.ds(0, nslots * CHUNK_M)])
        hs = []
        for b in range(nslots):
            base = base_g + b * CHUNK_M
            if not first:
                pltpu.make_async_copy(gb[b], acc.at[pl.ds(0, CHUNK_M)],
                                      ssem[b]).wait()
            hi = pltpu.async_copy(src_hbm.at[pl.ds(base, CHUNK_M)], ss[b],
                                  isem[b])
            hg = pltpu.async_copy(nf_hbm.at[gd.at[pl.ds(b * CHUNK_M, CHUNK_M)]],
                                  gb[b], gsem[b])
            he = pltpu.async_copy(ef_hbm.at[pl.ds(base, CHUNK_M)], eb[b],
                                  esem[b])
            hs.append((hi, hg, he))
        for b in range(nslots):
            hi, hg, he = hs[b]
            hg.wait()
            he.wait()
            _relu_add_rows(gb[b], eb[b])
            hi.wait()
            pltpu.async_copy(gb[b], acc.at[ss[b]], ssem[b], add=True)

    mgroup(base_w, True)
    lax.fori_loop(1, NGRP_M,
                  lambda g, carry: (mgroup(base_w + g * MGRP, False), carry)[1],
                  0)
    mgroup(base_w + NGRP_M * MGRP, False, nslots=NCHUNK_M - NGRP_M * NBUF_M)
    for b in range(NBUF_M):
        pltpu.make_async_copy(gb[b], acc.at[pl.ds(0, CHUNK_M)], ssem[b]).wait()

    plsc.subcore_barrier()
    sl = pl.ds(s * ZROWS, ZROWS)
    pltpu.sync_copy(acc.at[sl], out_hbm.at[c, sl])

    @pl.when(s == NS - 1)
    def _():
        rsl = pl.ds(REM_BASE, REM)
        pltpu.sync_copy(acc.at[rsl], out_hbm.at[c, rsl])


_GATHER_SCRATCH = (
    [pltpu.VMEM((GGRP,), jnp.int32) for _ in range(2)]
    + [pltpu.VMEM((CHUNK, D), jnp.float32) for _ in range(2 * NBUF_G)]
    + [pltpu.SemaphoreType.DMA for _ in range(4 * NBUF_G)]
)


@functools.partial(
    pl.kernel,
    out_type=(
        jax.ShapeDtypeStruct((E, D), jnp.float32),
        jax.ShapeDtypeStruct((E, D), jnp.float32),
    ),
    mesh=_MESH,
    scratch_types=_GATHER_SCRATCH,
)
def _sc_edge_gather(nf_hbm, src_hbm, dst_hbm, gi_hbm, gj_hbm, *sc):
    isrc = sc[0]
    idst = sc[1]
    gbi = sc[2:2 + NBUF_G]
    gbj = sc[7:7 + NBUF_G]
    gsi = sc[12:12 + NBUF_G]
    gsj = sc[17:17 + NBUF_G]
    wsi = sc[22:22 + NBUF_G]
    wsj = sc[27:27 + NBUF_G]

    c = lax.axis_index("c")
    s = lax.axis_index("s")
    wid = s * NC + c
    base_w = wid * EPW

    def group(base_g, first):
        pltpu.sync_copy(src_hbm.at[pl.ds(base_g, GGRP)], isrc)
        pltpu.sync_copy(dst_hbm.at[pl.ds(base_g, GGRP)], idst)
        hs = []
        for b in range(NBUF_G):
            if not first:
                pltpu.make_async_copy(gbi[b], gi_hbm.at[pl.ds(0, CHUNK)],
                                      wsi[b]).wait()
                pltpu.make_async_copy(gbj[b], gj_hbm.at[pl.ds(0, CHUNK)],
                                      wsj[b]).wait()
            h1 = pltpu.async_copy(nf_hbm.at[isrc.at[pl.ds(b * CHUNK, CHUNK)]],
                                  gbi[b], gsi[b])
            h2 = pltpu.async_copy(nf_hbm.at[idst.at[pl.ds(b * CHUNK, CHUNK)]],
                                  gbj[b], gsj[b])
            hs.append((h1, h2))
        for b in range(NBUF_G):
            h1, h2 = hs[b]
            sl = pl.ds(base_g + b * CHUNK, CHUNK)
            h1.wait()
            pltpu.async_copy(gbi[b], gi_hbm.at[sl], wsi[b])
            h2.wait()
            pltpu.async_copy(gbj[b], gj_hbm.at[sl], wsj[b])

    group(base_w, True)
    lax.fori_loop(1, NGRP_G,
                  lambda g, carry: (group(base_w + g * GGRP, False), carry)[1],
                  0)
    for b in range(NBUF_G):
        pltpu.make_async_copy(gbi[b], gi_hbm.at[pl.ds(0, CHUNK)], wsi[b]).wait()
        pltpu.make_async_copy(gbj[b], gj_hbm.at[pl.ds(0, CHUNK)], wsj[b]).wait()


def _ln(x, g, b):
    m = jnp.mean(x, axis=-1, keepdims=True)
    xc = x - m
    v = jnp.mean(xc * xc, axis=-1, keepdims=True)
    return xc * jax.lax.rsqrt(v + 1e-5) * g + b


BN = 1000


def _node_mlp_body(nf, p0, p1, eps, w1, b1, lg1, lb1, w2, b2, bng, bnb,
                   out_ln, out_relu):
    h = (1.0 + eps[0, 0]) * nf[...] + p0[...] + p1[...]
    t = jnp.dot(h, w1[...], preferred_element_type=jnp.float32) + b1[...]
    t = _ln(t, lg1[...], lb1[...])
    t = jnp.maximum(t, 0.0)
    u = jnp.dot(t, w2[...], preferred_element_type=jnp.float32) + b2[...]
    y = _ln(u, bng[...], bnb[...])
    out_ln[...] = y
    out_relu[...] = jnp.maximum(y, 0.0)


def _tc_node_mlp(nf, p0, p1, eps, w1, b1, lg1, lb1, w2, b2, bng, bnb):
    grid = (N // BN,)
    row_spec = pl.BlockSpec((BN, D), lambda i: (i, 0))
    full = lambda shape: pl.BlockSpec(shape, lambda i: (0, 0))
    return pl.pallas_call(
        _node_mlp_body,
        grid=grid,
        in_specs=[
            row_spec, row_spec, row_spec,
            full((1, 1)),
            full((D, 2 * D)), full((1, 2 * D)), full((1, 2 * D)), full((1, 2 * D)),
            full((2 * D, D)), full((1, D)), full((1, D)), full((1, D)),
        ],
        out_specs=[row_spec, row_spec],
        out_shape=[
            jax.ShapeDtypeStruct((N, D), jnp.float32),
            jax.ShapeDtypeStruct((N, D), jnp.float32),
        ],
    )(nf, p0, p1, eps, w1, b1, lg1, lb1, w2, b2, bng, bnb)


BM = 1280


def _ln_fast(x, g, b):
    m = jnp.mean(x, axis=-1, keepdims=True)
    m2 = jnp.mean(x * x, axis=-1, keepdims=True)
    rs = jax.lax.rsqrt(jnp.maximum(m2 - m * m, 0.0) + 1e-5)
    return (x - m) * rs * g + b


def _edge_mlp_body(gi, gj, ef, wa, wb, wc, b1, lg, lb, w2, b2, out):
    bf = jnp.bfloat16
    efv = ef[...]
    s = (gi[...] + gj[...]).astype(bf)
    a = jnp.abs(gi[...] - gj[...]).astype(bf)
    t = jnp.dot(s, wa[...], preferred_element_type=jnp.float32)
    t += jnp.dot(a, wb[...], preferred_element_type=jnp.float32)
    t += jnp.dot(efv.astype(bf), wc[...], preferred_element_type=jnp.float32)
    t += b1[...]
    t = _ln_fast(t, lg[...], lb[...])
    t = jnp.maximum(t, 0.0).astype(bf)
    u = jnp.dot(t, w2[...], preferred_element_type=jnp.float32) + b2[...]
    out[...] = u + efv


def _tc_edge_mlp(gi, gj, ef, wa, wb, wc, b1, lg, lb, w2, b2):
    grid = (E // BM,)
    full = lambda shape: pl.BlockSpec(shape, lambda i: (0, 0))
    row_spec = pl.BlockSpec((BM, D), lambda i: (i, 0))
    return pl.pallas_call(
        _edge_mlp_body,
        grid=grid,
        in_specs=[
            row_spec, row_spec, row_spec,
            full((D, 3 * D)), full((D, 3 * D)), full((D, 3 * D)),
            full((1, 3 * D)), full((1, 3 * D)), full((1, 3 * D)),
            full((3 * D, D)), full((1, D)),
        ],
        out_specs=row_spec,
        out_shape=jax.ShapeDtypeStruct((E, D), jnp.float32),
    )(gi, gj, ef,
      wa.astype(jnp.bfloat16), wb.astype(jnp.bfloat16), wc.astype(jnp.bfloat16),
      b1, lg, lb, w2.astype(jnp.bfloat16), b2)


def kernel(node_feats, edge_feats, edge_index, params):
    src = edge_index[0]
    dst = edge_index[1]
    nf = node_feats
    ef = edge_feats
    nf_ln = node_feats
    num_layers = len(params)
    for l in range(num_layers):
        p = params["layer%d" % l]
        parts = _sc_message(nf, ef, src, dst)
        nf_ln, nf_relu = _tc_node_mlp(
            nf, parts[0], parts[1],
            p["eps"].reshape(1, 1),
            p["cW1"], p["cb1"].reshape(1, -1),
            p["cln_g"].reshape(1, -1), p["cln_b"].reshape(1, -1),
            p["cW2"], p["cb2"].reshape(1, -1),
            p["bn_g"].reshape(1, -1), p["bn_b"].reshape(1, -1),
        )
        gi, gj = _sc_edge_gather(nf_ln, src, dst)
        ef = _tc_edge_mlp(
            gi, gj, ef,
            p["eW1"][:D], p["eW1"][D:2 * D], p["eW1"][2 * D:],
            p["eb1"].reshape(1, -1),
            p["eln_g"].reshape(1, -1), p["eln_b"].reshape(1, -1),
            p["eW2"], p["eb2"].reshape(1, -1),
        )
        nf = nf_relu
    return nf_ln, ef


# speedup vs baseline: 1.4360x; 1.1328x over previous
import functools

import jax
import jax.numpy as jnp
from jax import lax
from jax.experimental import pallas as pl
from jax.experimental.pallas import tpu as pltpu
from jax.experimental.pallas import tpu_sc as plsc

N = 10000
D = 128
NC = 2
NS = 16
NW = NC * NS
ZROWS = 624
REM_BASE = NS * ZROWS
REM = N - REM_BASE
ZBUF = 16

CHUNK_G = 40
NBUF_G = 5

CHUNK_M = 40
NBUF_M = 4

_MESH = plsc.VectorSubcoreMesh(
    core_axis_name="c", subcore_axis_name="s", num_cores=NC, num_subcores=NS)


def _relu_add_rows(g_ref, e_ref):
    def row(r, carry):
        for k in range(D // 16):
            sl = pl.ds(k * 16, 16)
            g_ref[r, sl] = jnp.maximum(g_ref[r, sl] + e_ref[r, sl], 0.0)
        return carry
    lax.fori_loop(0, CHUNK_M, row, 0, unroll=2)


_MSG_SCRATCH = (
    [pltpu.VMEM_SHARED((N, D), jnp.float32)]
    + [pltpu.VMEM((NBUF_M * CHUNK_M,), jnp.int32)]
    + [pltpu.VMEM((CHUNK_M,), jnp.int32) for _ in range(NBUF_M)]
    + [pltpu.VMEM((CHUNK_M, D), jnp.float32) for _ in range(NBUF_M)]
    + [pltpu.VMEM((CHUNK_M, D), jnp.float32) for _ in range(NBUF_M)]
    + [pltpu.VMEM((ZBUF, D), jnp.float32)]
    + [pltpu.SemaphoreType.DMA for _ in range(4 * NBUF_M)]
)


@functools.cache
def _make_sc_message(nedges):
    epw = nedges // NW
    nchunk = epw // CHUNK_M
    mgrp = NBUF_M * CHUNK_M
    ngrp = nchunk // NBUF_M
    tail = nchunk - ngrp * NBUF_M

    @functools.partial(
        pl.kernel,
        out_type=jax.ShapeDtypeStruct((NC, N, D), jnp.float32),
        mesh=_MESH,
        scratch_types=_MSG_SCRATCH,
    )
    def sc_message(nf_hbm, ef_hbm, src_hbm, dst_hbm, out_hbm, *sc):
        acc = sc[0]
        gd = sc[1]
        ss = sc[2:2 + NBUF_M]
        gb = sc[2 + NBUF_M:2 + 2 * NBUF_M]
        eb = sc[2 + 2 * NBUF_M:2 + 3 * NBUF_M]
        zbuf = sc[2 + 3 * NBUF_M]
        sems = sc[3 + 3 * NBUF_M:]
        isem = sems[0:NBUF_M]
        gsem = sems[NBUF_M:2 * NBUF_M]
        esem = sems[2 * NBUF_M:3 * NBUF_M]
        ssem = sems[3 * NBUF_M:4 * NBUF_M]

        c = lax.axis_index("c")
        s = lax.axis_index("s")
        wid = s * NC + c
        base_w = wid * epw

        def zrow(r, carry):
            for k in range(D // 16):
                zbuf[r, pl.ds(k * 16, 16)] = jnp.zeros((16,), jnp.float32)
            return carry
        lax.fori_loop(0, ZBUF, zrow, 0)
        for j in range(ZROWS // ZBUF):
            pltpu.sync_copy(zbuf, acc.at[pl.ds(s * ZROWS + j * ZBUF, ZBUF)])

        @pl.when(s == NS - 1)
        def _():
            pltpu.sync_copy(zbuf.at[pl.ds(0, REM)],
                            acc.at[pl.ds(REM_BASE, REM)])
        plsc.subcore_barrier()

        def mgroup(base_g, first, nslots=NBUF_M):
            pltpu.sync_copy(dst_hbm.at[pl.ds(base_g, nslots * CHUNK_M)],
                            gd.at[pl.ds(0, nslots * CHUNK_M)])
            hs = []
            for b in range(nslots):
                base = base_g + b * CHUNK_M
                if not first:
                    pltpu.make_async_copy(gb[b], acc.at[pl.ds(0, CHUNK_M)],
                                          ssem[b]).wait()
                hi = pltpu.async_copy(src_hbm.at[pl.ds(base, CHUNK_M)], ss[b],
                                      isem[b])
                hg = pltpu.async_copy(
                    nf_hbm.at[gd.at[pl.ds(b * CHUNK_M, CHUNK_M)]],
                    gb[b], gsem[b])
                he = pltpu.async_copy(ef_hbm.at[pl.ds(base, CHUNK_M)], eb[b],
                                      esem[b])
                hs.append((hi, hg, he))
            for b in range(nslots):
                hi, hg, he = hs[b]
                hg.wait()
                he.wait()
                _relu_add_rows(gb[b], eb[b])
                hi.wait()
                pltpu.async_copy(gb[b], acc.at[ss[b]], ssem[b], add=True)

        mgroup(base_w, True)
        lax.fori_loop(
            1, ngrp,
            lambda g, carry: (mgroup(base_w + g * mgrp, False), carry)[1],
            0)
        if tail:
            mgroup(base_w + ngrp * mgrp, False, nslots=tail)
        for b in range(NBUF_M):
            pltpu.make_async_copy(gb[b], acc.at[pl.ds(0, CHUNK_M)],
                                  ssem[b]).wait()

        plsc.subcore_barrier()
        sl = pl.ds(s * ZROWS, ZROWS)
        pltpu.sync_copy(acc.at[sl], out_hbm.at[c, sl])

        @pl.when(s == NS - 1)
        def _():
            rsl = pl.ds(REM_BASE, REM)
            pltpu.sync_copy(acc.at[rsl], out_hbm.at[c, rsl])

    return sc_message


_GATHER_SCRATCH = (
    [pltpu.VMEM((NBUF_G * CHUNK_G,), jnp.int32) for _ in range(2)]
    + [pltpu.VMEM((CHUNK_G, D), jnp.float32) for _ in range(2 * NBUF_G)]
    + [pltpu.SemaphoreType.DMA for _ in range(4 * NBUF_G)]
)


@functools.cache
def _make_sc_gather(nedges):
    epw = nedges // NW
    ggrp = NBUF_G * CHUNK_G
    ngrp = epw // ggrp

    @functools.partial(
        pl.kernel,
        out_type=(
            jax.ShapeDtypeStruct((nedges, D), jnp.float32),
            jax.ShapeDtypeStruct((nedges, D), jnp.float32),
        ),
        mesh=_MESH,
        scratch_types=_GATHER_SCRATCH,
    )
    def sc_edge_gather(nf_hbm, src_hbm, dst_hbm, gi_hbm, gj_hbm, *sc):
        isrc = sc[0]
        idst = sc[1]
        gbi = sc[2:2 + NBUF_G]
        gbj = sc[2 + NBUF_G:2 + 2 * NBUF_G]
        sems = sc[2 + 2 * NBUF_G:]
        gsi = sems[0:NBUF_G]
        gsj = sems[NBUF_G:2 * NBUF_G]
        wsi = sems[2 * NBUF_G:3 * NBUF_G]
        wsj = sems[3 * NBUF_G:4 * NBUF_G]

        c = lax.axis_index("c")
        s = lax.axis_index("s")
        wid = s * NC + c
        base_w = wid * epw

        def group(base_g, first):
            pltpu.sync_copy(src_hbm.at[pl.ds(base_g, ggrp)], isrc)
            pltpu.sync_copy(dst_hbm.at[pl.ds(base_g, ggrp)], idst)
            hs = []
            for b in range(NBUF_G):
                if not first:
                    pltpu.make_async_copy(gbi[b], gi_hbm.at[pl.ds(0, CHUNK_G)],
                                          wsi[b]).wait()
                    pltpu.make_async_copy(gbj[b], gj_hbm.at[pl.ds(0, CHUNK_G)],
                                          wsj[b]).wait()
                h1 = pltpu.async_copy(
                    nf_hbm.at[isrc.at[pl.ds(b * CHUNK_G, CHUNK_G)]],
                    gbi[b], gsi[b])
                h2 = pltpu.async_copy(
                    nf_hbm.at[idst.at[pl.ds(b * CHUNK_G, CHUNK_G)]],
                    gbj[b], gsj[b])
                hs.append((h1, h2))
            for b in range(NBUF_G):
                h1, h2 = hs[b]
                sl = pl.ds(base_g + b * CHUNK_G, CHUNK_G)
                h1.wait()
                pltpu.async_copy(gbi[b], gi_hbm.at[sl], wsi[b])
                h2.wait()
                pltpu.async_copy(gbj[b], gj_hbm.at[sl], wsj[b])

        group(base_w, True)
        lax.fori_loop(
            1, ngrp,
            lambda g, carry: (group(base_w + g * ggrp, False), carry)[1],
            0)
        for b in range(NBUF_G):
            pltpu.make_async_copy(gbi[b], gi_hbm.at[pl.ds(0, CHUNK_G)],
                                  wsi[b]).wait()
            pltpu.make_async_copy(gbj[b], gj_hbm.at[pl.ds(0, CHUNK_G)],
                                  wsj[b]).wait()

    return sc_edge_gather


def _ln(x, g, b):
    m = jnp.mean(x, axis=-1, keepdims=True)
    xc = x - m
    v = jnp.mean(xc * xc, axis=-1, keepdims=True)
    return xc * jax.lax.rsqrt(v + 1e-5) * g + b


def _ln_fast(x, g, b):
    m = jnp.mean(x, axis=-1, keepdims=True)
    m2 = jnp.mean(x * x, axis=-1, keepdims=True)
    rs = jax.lax.rsqrt(jnp.maximum(m2 - m * m, 0.0) + 1e-5)
    return (x - m) * rs * g + b


BN = 1000


def _node_mlp_body(nf, p0, p1, p2, p3, eps, w1, b1, lg1, lb1, w2, b2,
                   bng, bnb, out_ln, out_relu):
    h = ((1.0 + eps[0, 0]) * nf[...] + (p0[...] + p1[...])
         + (p2[...] + p3[...]))
    t = jnp.dot(h, w1[...], preferred_element_type=jnp.float32) + b1[...]
    t = _ln(t, lg1[...], lb1[...])
    t = jnp.maximum(t, 0.0)
    u = jnp.dot(t, w2[...], preferred_element_type=jnp.float32) + b2[...]
    y = _ln(u, bng[...], bnb[...])
    out_ln[...] = y
    out_relu[...] = jnp.maximum(y, 0.0)


def _tc_node_mlp(nf, p0, p1, p2, p3, eps, w1, b1, lg1, lb1, w2, b2, bng, bnb):
    grid = (N // BN,)
    row_spec = pl.BlockSpec((BN, D), lambda i: (i, 0))
    full = lambda shape: pl.BlockSpec(shape, lambda i: (0, 0))
    return pl.pallas_call(
        _node_mlp_body,
        grid=grid,
        in_specs=[
            row_spec, row_spec, row_spec, row_spec, row_spec,
            full((1, 1)),
            full((D, 2 * D)), full((1, 2 * D)), full((1, 2 * D)), full((1, 2 * D)),
            full((2 * D, D)), full((1, D)), full((1, D)), full((1, D)),
        ],
        out_specs=[row_spec, row_spec],
        out_shape=[
            jax.ShapeDtypeStruct((N, D), jnp.float32),
            jax.ShapeDtypeStruct((N, D), jnp.float32),
        ],
    )(nf, p0, p1, p2, p3, eps, w1, b1, lg1, lb1, w2, b2, bng, bnb)


BM = 1280


def _edge_mlp_body(gi, gj, ef, wa, wb, wc, b1, lg, lb, w2, b2, out):
    bf = jnp.bfloat16
    efv = ef[...]
    s = (gi[...] + gj[...]).astype(bf)
    a = jnp.abs(gi[...] - gj[...]).astype(bf)
    t = jnp.dot(s, wa[...], preferred_element_type=jnp.float32)
    t += jnp.dot(a, wb[...], preferred_element_type=jnp.float32)
    t += jnp.dot(efv.astype(bf), wc[...], preferred_element_type=jnp.float32)
    t += b1[...]
    t = _ln_fast(t, lg[...], lb[...])
    t = jnp.maximum(t, 0.0).astype(bf)
    u = jnp.dot(t, w2[...], preferred_element_type=jnp.float32) + b2[...]
    out[...] = u + efv


def _tc_edge_mlp(gi, gj, ef, wa, wb, wc, b1, lg, lb, w2, b2):
    nedges = gi.shape[0]
    grid = (nedges // BM,)
    full = lambda shape: pl.BlockSpec(shape, lambda i: (0, 0))
    row_spec = pl.BlockSpec((BM, D), lambda i: (i, 0))
    return pl.pallas_call(
        _edge_mlp_body,
        grid=grid,
        in_specs=[
            row_spec, row_spec, row_spec,
            full((D, 3 * D)), full((D, 3 * D)), full((D, 3 * D)),
            full((1, 3 * D)), full((1, 3 * D)), full((1, 3 * D)),
            full((3 * D, D)), full((1, D)),
        ],
        out_specs=row_spec,
        out_shape=jax.ShapeDtypeStruct((nedges, D), jnp.float32),
    )(gi, gj, ef,
      wa.astype(jnp.bfloat16), wb.astype(jnp.bfloat16), wc.astype(jnp.bfloat16),
      b1, lg, lb, w2.astype(jnp.bfloat16), b2)


def kernel(node_feats, edge_feats, edge_index, params):
    E = edge_feats.shape[0]
    E2 = E // 2
    src0, src1 = edge_index[0, :E2], edge_index[0, E2:]
    dst0, dst1 = edge_index[1, :E2], edge_index[1, E2:]
    ef0, ef1 = edge_feats[:E2], edge_feats[E2:]
    sc_message = _make_sc_message(E2)
    sc_gather = _make_sc_gather(E2)
    nf = node_feats
    nf_ln = node_feats
    num_layers = len(params)
    for l in range(num_layers):
        p = params["layer%d" % l]
        pa = sc_message(nf, ef0, src0, dst0)
        pb = sc_message(nf, ef1, src1, dst1)
        nf_ln, nf_relu = _tc_node_mlp(
            nf, pa[0], pa[1], pb[0], pb[1],
            p["eps"].reshape(1, 1),
            p["cW1"], p["cb1"].reshape(1, -1),
            p["cln_g"].reshape(1, -1), p["cln_b"].reshape(1, -1),
            p["cW2"], p["cb2"].reshape(1, -1),
            p["bn_g"].reshape(1, -1), p["bn_b"].reshape(1, -1),
        )
        ew = (p["eW1"][:D], p["eW1"][D:2 * D], p["eW1"][2 * D:],
              p["eb1"].reshape(1, -1),
              p["eln_g"].reshape(1, -1), p["eln_b"].reshape(1, -1),
              p["eW2"], p["eb2"].reshape(1, -1))
        gi0, gj0 = sc_gather(nf_ln, src0, dst0)
        gi1, gj1 = sc_gather(nf_ln, src1, dst1)
        ef0 = _tc_edge_mlp(gi0, gj0, ef0, *ew)
        ef1 = _tc_edge_mlp(gi1, gj1, ef1, *ew)
        nf = nf_relu
    return nf_ln, jnp.concatenate([ef0, ef1], axis=0)


# speedup vs baseline: 1.4717x; 1.0248x over previous
import functools

import jax
import jax.numpy as jnp
from jax import lax
from jax.experimental import pallas as pl
from jax.experimental.pallas import tpu as pltpu
from jax.experimental.pallas import tpu_sc as plsc

N = 10000
D = 128
NC = 2
NS = 16
NW = NC * NS
ZROWS = 624
REM_BASE = NS * ZROWS
REM = N - REM_BASE
ZBUF = 16

CHUNK_G = 40
NBUF_G = 5

CHUNK_M = 40
NBUF_M = 4

_MESH = plsc.VectorSubcoreMesh(
    core_axis_name="c", subcore_axis_name="s", num_cores=NC, num_subcores=NS)


def _relu_add_rows(g_ref, e_ref, pre_relu=False):
    def row(r, carry):
        for k in range(D // 16):
            sl = pl.ds(k * 16, 16)
            g = g_ref[r, sl]
            if pre_relu:
                g = jnp.maximum(g, 0.0)
            g_ref[r, sl] = jnp.maximum(g + e_ref[r, sl], 0.0)
        return carry
    lax.fori_loop(0, CHUNK_M, row, 0, unroll=2)


_MSG_SCRATCH = (
    [pltpu.VMEM_SHARED((N, D), jnp.float32)]
    + [pltpu.VMEM((NBUF_M * CHUNK_M,), jnp.int32)]
    + [pltpu.VMEM((CHUNK_M,), jnp.int32) for _ in range(NBUF_M)]
    + [pltpu.VMEM((CHUNK_M, D), jnp.float32) for _ in range(NBUF_M)]
    + [pltpu.VMEM((CHUNK_M, D), jnp.float32) for _ in range(NBUF_M)]
    + [pltpu.VMEM((ZBUF, D), jnp.float32)]
    + [pltpu.SemaphoreType.DMA for _ in range(4 * NBUF_M)]
)


@functools.cache
def _make_sc_message(nedges, linear=False):
    epw = nedges // NW
    nchunk = epw // CHUNK_M
    mgrp = NBUF_M * CHUNK_M
    ngrp = nchunk // NBUF_M
    tail = nchunk - ngrp * NBUF_M

    def sc_message_body(*refs):
        if linear:
            nf_hbm, ef_hbm, src_hbm, out_hbm = refs[:4]
            dst_hbm = None
            sc = refs[4:]
        else:
            nf_hbm, ef_hbm, src_hbm, dst_hbm, out_hbm = refs[:5]
            sc = refs[5:]
        acc = sc[0]
        gd = sc[1]
        ss = sc[2:2 + NBUF_M]
        gb = sc[2 + NBUF_M:2 + 2 * NBUF_M]
        eb = sc[2 + 2 * NBUF_M:2 + 3 * NBUF_M]
        zbuf = sc[2 + 3 * NBUF_M]
        sems = sc[3 + 3 * NBUF_M:]
        isem = sems[0:NBUF_M]
        gsem = sems[NBUF_M:2 * NBUF_M]
        esem = sems[2 * NBUF_M:3 * NBUF_M]
        ssem = sems[3 * NBUF_M:4 * NBUF_M]

        c = lax.axis_index("c")
        s = lax.axis_index("s")
        wid = s * NC + c
        base_w = wid * epw

        def zrow(r, carry):
            for k in range(D // 16):
                zbuf[r, pl.ds(k * 16, 16)] = jnp.zeros((16,), jnp.float32)
            return carry
        lax.fori_loop(0, ZBUF, zrow, 0)
        for j in range(ZROWS // ZBUF):
            pltpu.sync_copy(zbuf, acc.at[pl.ds(s * ZROWS + j * ZBUF, ZBUF)])

        @pl.when(s == NS - 1)
        def _():
            pltpu.sync_copy(zbuf.at[pl.ds(0, REM)],
                            acc.at[pl.ds(REM_BASE, REM)])
        plsc.subcore_barrier()

        def mgroup(base_g, first, nslots=NBUF_M):
            if not linear:
                pltpu.sync_copy(dst_hbm.at[pl.ds(base_g, nslots * CHUNK_M)],
                                gd.at[pl.ds(0, nslots * CHUNK_M)])
            hs = []
            for b in range(nslots):
                base = base_g + b * CHUNK_M
                if not first:
                    pltpu.make_async_copy(gb[b], acc.at[pl.ds(0, CHUNK_M)],
                                          ssem[b]).wait()
                hi = pltpu.async_copy(src_hbm.at[pl.ds(base, CHUNK_M)], ss[b],
                                      isem[b])
                if linear:
                    hg = pltpu.async_copy(nf_hbm.at[pl.ds(base, CHUNK_M)],
                                          gb[b], gsem[b])
                else:
                    hg = pltpu.async_copy(
                        nf_hbm.at[gd.at[pl.ds(b * CHUNK_M, CHUNK_M)]],
                        gb[b], gsem[b])
                he = pltpu.async_copy(ef_hbm.at[pl.ds(base, CHUNK_M)], eb[b],
                                      esem[b])
                hs.append((hi, hg, he))
            for b in range(nslots):
                hi, hg, he = hs[b]
                hg.wait()
                he.wait()
                _relu_add_rows(gb[b], eb[b], pre_relu=linear)
                hi.wait()
                pltpu.async_copy(gb[b], acc.at[ss[b]], ssem[b], add=True)

        mgroup(base_w, True)
        lax.fori_loop(
            1, ngrp,
            lambda g, carry: (mgroup(base_w + g * mgrp, False), carry)[1],
            0)
        if tail:
            mgroup(base_w + ngrp * mgrp, False, nslots=tail)
        for b in range(NBUF_M):
            pltpu.make_async_copy(gb[b], acc.at[pl.ds(0, CHUNK_M)],
                                  ssem[b]).wait()

        plsc.subcore_barrier()
        sl = pl.ds(s * ZROWS, ZROWS)
        pltpu.sync_copy(acc.at[sl], out_hbm.at[c, sl])

        @pl.when(s == NS - 1)
        def _():
            rsl = pl.ds(REM_BASE, REM)
            pltpu.sync_copy(acc.at[rsl], out_hbm.at[c, rsl])

    return pl.kernel(
        sc_message_body,
        out_type=jax.ShapeDtypeStruct((NC, N, D), jnp.float32),
        mesh=_MESH,
        scratch_types=_MSG_SCRATCH,
    )


_GATHER_SCRATCH = (
    [pltpu.VMEM((NBUF_G * CHUNK_G,), jnp.int32) for _ in range(2)]
    + [pltpu.VMEM((CHUNK_G, D), jnp.float32) for _ in range(2 * NBUF_G)]
    + [pltpu.SemaphoreType.DMA for _ in range(4 * NBUF_G)]
)


@functools.cache
def _make_sc_gather(nedges):
    epw = nedges // NW
    ggrp = NBUF_G * CHUNK_G
    ngrp = epw // ggrp

    @functools.partial(
        pl.kernel,
        out_type=(
            jax.ShapeDtypeStruct((nedges, D), jnp.float32),
            jax.ShapeDtypeStruct((nedges, D), jnp.float32),
        ),
        mesh=_MESH,
        scratch_types=_GATHER_SCRATCH,
    )
    def sc_edge_gather(nf_hbm, src_hbm, dst_hbm, gi_hbm, gj_hbm, *sc):
        isrc = sc[0]
        idst = sc[1]
        gbi = sc[2:2 + NBUF_G]
        gbj = sc[2 + NBUF_G:2 + 2 * NBUF_G]
        sems = sc[2 + 2 * NBUF_G:]
        gsi = sems[0:NBUF_G]
        gsj = sems[NBUF_G:2 * NBUF_G]
        wsi = sems[2 * NBUF_G:3 * NBUF_G]
        wsj = sems[3 * NBUF_G:4 * NBUF_G]

        c = lax.axis_index("c")
        s = lax.axis_index("s")
        wid = s * NC + c
        base_w = wid * epw

        def group(base_g, first):
            pltpu.sync_copy(src_hbm.at[pl.ds(base_g, ggrp)], isrc)
            pltpu.sync_copy(dst_hbm.at[pl.ds(base_g, ggrp)], idst)
            hs = []
            for b in range(NBUF_G):
                if not first:
                    pltpu.make_async_copy(gbi[b], gi_hbm.at[pl.ds(0, CHUNK_G)],
                                          wsi[b]).wait()
                    pltpu.make_async_copy(gbj[b], gj_hbm.at[pl.ds(0, CHUNK_G)],
                                          wsj[b]).wait()
                h1 = pltpu.async_copy(
                    nf_hbm.at[isrc.at[pl.ds(b * CHUNK_G, CHUNK_G)]],
                    gbi[b], gsi[b])
                h2 = pltpu.async_copy(
                    nf_hbm.at[idst.at[pl.ds(b * CHUNK_G, CHUNK_G)]],
                    gbj[b], gsj[b])
                hs.append((h1, h2))
            for b in range(NBUF_G):
                h1, h2 = hs[b]
                sl = pl.ds(base_g + b * CHUNK_G, CHUNK_G)
                h1.wait()
                pltpu.async_copy(gbi[b], gi_hbm.at[sl], wsi[b])
                h2.wait()
                pltpu.async_copy(gbj[b], gj_hbm.at[sl], wsj[b])

        group(base_w, True)
        lax.fori_loop(
            1, ngrp,
            lambda g, carry: (group(base_w + g * ggrp, False), carry)[1],
            0)
        for b in range(NBUF_G):
            pltpu.make_async_copy(gbi[b], gi_hbm.at[pl.ds(0, CHUNK_G)],
                                  wsi[b]).wait()
            pltpu.make_async_copy(gbj[b], gj_hbm.at[pl.ds(0, CHUNK_G)],
                                  wsj[b]).wait()

    return sc_edge_gather


def _ln(x, g, b):
    m = jnp.mean(x, axis=-1, keepdims=True)
    xc = x - m
    v = jnp.mean(xc * xc, axis=-1, keepdims=True)
    return xc * jax.lax.rsqrt(v + 1e-5) * g + b


def _ln_fast(x, g, b):
    m = jnp.mean(x, axis=-1, keepdims=True)
    m2 = jnp.mean(x * x, axis=-1, keepdims=True)
    rs = jax.lax.rsqrt(jnp.maximum(m2 - m * m, 0.0) + 1e-5)
    return (x - m) * rs * g + b


BN = 1000


def _node_mlp_body(nf, p0, p1, p2, p3, eps, w1, b1, lg1, lb1, w2, b2,
                   bng, bnb, out_ln, out_relu):
    h = ((1.0 + eps[0, 0]) * nf[...] + (p0[...] + p1[...])
         + (p2[...] + p3[...]))
    t = jnp.dot(h, w1[...], preferred_element_type=jnp.float32) + b1[...]
    t = _ln(t, lg1[...], lb1[...])
    t = jnp.maximum(t, 0.0)
    u = jnp.dot(t, w2[...], preferred_element_type=jnp.float32) + b2[...]
    y = _ln(u, bng[...], bnb[...])
    out_ln[...] = y
    out_relu[...] = jnp.maximum(y, 0.0)


def _tc_node_mlp(nf, p0, p1, p2, p3, eps, w1, b1, lg1, lb1, w2, b2, bng, bnb):
    grid = (N // BN,)
    row_spec = pl.BlockSpec((BN, D), lambda i: (i, 0))
    full = lambda shape: pl.BlockSpec(shape, lambda i: (0, 0))
    return pl.pallas_call(
        _node_mlp_body,
        grid=grid,
        in_specs=[
            row_spec, row_spec, row_spec, row_spec, row_spec,
            full((1, 1)),
            full((D, 2 * D)), full((1, 2 * D)), full((1, 2 * D)), full((1, 2 * D)),
            full((2 * D, D)), full((1, D)), full((1, D)), full((1, D)),
        ],
        out_specs=[row_spec, row_spec],
        out_shape=[
            jax.ShapeDtypeStruct((N, D), jnp.float32),
            jax.ShapeDtypeStruct((N, D), jnp.float32),
        ],
    )(nf, p0, p1, p2, p3, eps, w1, b1, lg1, lb1, w2, b2, bng, bnb)


BM = 1280


def _edge_mlp_body(gi, gj, ef, wa, wb, wc, b1, lg, lb, w2, b2, out):
    bf = jnp.bfloat16
    efv = ef[...]
    s = (gi[...] + gj[...]).astype(bf)
    a = jnp.abs(gi[...] - gj[...]).astype(bf)
    t = jnp.dot(s, wa[...], preferred_element_type=jnp.float32)
    t += jnp.dot(a, wb[...], preferred_element_type=jnp.float32)
    t += jnp.dot(efv.astype(bf), wc[...], preferred_element_type=jnp.float32)
    t += b1[...]
    t = _ln_fast(t, lg[...], lb[...])
    t = jnp.maximum(t, 0.0).astype(bf)
    u = jnp.dot(t, w2[...], preferred_element_type=jnp.float32) + b2[...]
    out[...] = u + efv


def _tc_edge_mlp(gi, gj, ef, wa, wb, wc, b1, lg, lb, w2, b2):
    nedges = gi.shape[0]
    grid = (nedges // BM,)
    full = lambda shape: pl.BlockSpec(shape, lambda i: (0, 0))
    row_spec = pl.BlockSpec((BM, D), lambda i: (i, 0))
    return pl.pallas_call(
        _edge_mlp_body,
        grid=grid,
        in_specs=[
            row_spec, row_spec, row_spec,
            full((D, 3 * D)), full((D, 3 * D)), full((D, 3 * D)),
            full((1, 3 * D)), full((1, 3 * D)), full((1, 3 * D)),
            full((3 * D, D)), full((1, D)),
        ],
        out_specs=row_spec,
        out_shape=jax.ShapeDtypeStruct((nedges, D), jnp.float32),
    )(gi, gj, ef,
      wa.astype(jnp.bfloat16), wb.astype(jnp.bfloat16), wc.astype(jnp.bfloat16),
      b1, lg, lb, w2.astype(jnp.bfloat16), b2)


def kernel(node_feats, edge_feats, edge_index, params):
    E = edge_feats.shape[0]
    E2 = E // 2
    src0, src1 = edge_index[0, :E2], edge_index[0, E2:]
    dst0, dst1 = edge_index[1, :E2], edge_index[1, E2:]
    ef0, ef1 = edge_feats[:E2], edge_feats[E2:]
    sc_message = _make_sc_message(E2)
    sc_message_lin = _make_sc_message(E2, linear=True)
    sc_gather = _make_sc_gather(E2)
    nf = node_feats
    nf_ln = node_feats
    gj0 = gj1 = None
    num_layers = len(params)
    for l in range(num_layers):
        p = params["layer%d" % l]
        if l == 0:
            pa = sc_message(nf, ef0, src0, dst0)
            pb = sc_message(nf, ef1, src1, dst1)
        else:
            pa = sc_message_lin(gj0, ef0, src0)
            pb = sc_message_lin(gj1, ef1, src1)
        nf_ln, nf_relu = _tc_node_mlp(
            nf, pa[0], pa[1], pb[0], pb[1],
            p["eps"].reshape(1, 1),
            p["cW1"], p["cb1"].reshape(1, -1),
            p["cln_g"].reshape(1, -1), p["cln_b"].reshape(1, -1),
            p["cW2"], p["cb2"].reshape(1, -1),
            p["bn_g"].reshape(1, -1), p["bn_b"].reshape(1, -1),
        )
        ew = (p["eW1"][:D], p["eW1"][D:2 * D], p["eW1"][2 * D:],
              p["eb1"].reshape(1, -1),
              p["eln_g"].reshape(1, -1), p["eln_b"].reshape(1, -1),
              p["eW2"], p["eb2"].reshape(1, -1))
        gi0, gj0 = sc_gather(nf_ln, src0, dst0)
        gi1, gj1 = sc_gather(nf_ln, src1, dst1)
        ef0 = _tc_edge_mlp(gi0, gj0, ef0, *ew)
        ef1 = _tc_edge_mlp(gi1, gj1, ef1, *ew)
        nf = nf_relu
    return nf_ln, jnp.concatenate([ef0, ef1], axis=0)


# speedup vs baseline: 1.5028x; 1.0211x over previous
import functools

import jax
import jax.numpy as jnp
from jax import lax
from jax.experimental import pallas as pl
from jax.experimental.pallas import tpu as pltpu
from jax.experimental.pallas import tpu_sc as plsc

N = 10000
D = 128
NC = 2
NS = 16
NW = NC * NS
ZROWS = 624
REM_BASE = NS * ZROWS
REM = N - REM_BASE
ZBUF = 16

CHUNK_G = 40
NBUF_G = 5

CHUNK_M = 40
NBUF_M = 4

_MESH = plsc.VectorSubcoreMesh(
    core_axis_name="c", subcore_axis_name="s", num_cores=NC, num_subcores=NS)


def _relu_add_rows(g_ref, e_ref, pre_relu=False):
    def row(r, carry):
        for k in range(D // 16):
            sl = pl.ds(k * 16, 16)
            g = g_ref[r, sl]
            if pre_relu:
                g = jnp.maximum(g, 0.0)
            g_ref[r, sl] = jnp.maximum(g + e_ref[r, sl], 0.0)
        return carry
    lax.fori_loop(0, CHUNK_M, row, 0, unroll=2)


_MSG_SCRATCH = (
    [pltpu.VMEM_SHARED((N, D), jnp.float32)]
    + [pltpu.VMEM((NBUF_M * CHUNK_M,), jnp.int32)]
    + [pltpu.VMEM((CHUNK_M,), jnp.int32) for _ in range(NBUF_M)]
    + [pltpu.VMEM((CHUNK_M, D), jnp.float32) for _ in range(NBUF_M)]
    + [pltpu.VMEM((CHUNK_M, D), jnp.float32) for _ in range(NBUF_M)]
    + [pltpu.VMEM((ZBUF, D), jnp.float32)]
    + [pltpu.SemaphoreType.DMA for _ in range(4 * NBUF_M)]
)


@functools.cache
def _make_sc_message(nedges, linear=False, off=0, ef_off=0):
    epw = nedges // NW
    nchunk = epw // CHUNK_M
    mgrp = NBUF_M * CHUNK_M
    ngrp = nchunk // NBUF_M
    tail = nchunk - ngrp * NBUF_M

    def sc_message_body(*refs):
        if linear:
            nf_hbm, ef_hbm, src_hbm, out_hbm = refs[:4]
            dst_hbm = None
            sc = refs[4:]
        else:
            nf_hbm, ef_hbm, src_hbm, dst_hbm, out_hbm = refs[:5]
            sc = refs[5:]
        acc = sc[0]
        gd = sc[1]
        ss = sc[2:2 + NBUF_M]
        gb = sc[2 + NBUF_M:2 + 2 * NBUF_M]
        eb = sc[2 + 2 * NBUF_M:2 + 3 * NBUF_M]
        zbuf = sc[2 + 3 * NBUF_M]
        sems = sc[3 + 3 * NBUF_M:]
        isem = sems[0:NBUF_M]
        gsem = sems[NBUF_M:2 * NBUF_M]
        esem = sems[2 * NBUF_M:3 * NBUF_M]
        ssem = sems[3 * NBUF_M:4 * NBUF_M]

        c = lax.axis_index("c")
        s = lax.axis_index("s")
        wid = s * NC + c
        base_w = wid * epw

        def zrow(r, carry):
            for k in range(D // 16):
                zbuf[r, pl.ds(k * 16, 16)] = jnp.zeros((16,), jnp.float32)
            return carry
        lax.fori_loop(0, ZBUF, zrow, 0)
        for j in range(ZROWS // ZBUF):
            pltpu.sync_copy(zbuf, acc.at[pl.ds(s * ZROWS + j * ZBUF, ZBUF)])

        @pl.when(s == NS - 1)
        def _():
            pltpu.sync_copy(zbuf.at[pl.ds(0, REM)],
                            acc.at[pl.ds(REM_BASE, REM)])
        plsc.subcore_barrier()

        def mgroup(base_g, first, nslots=NBUF_M):
            if not linear:
                pltpu.sync_copy(
                    dst_hbm.at[pl.ds(off + base_g, nslots * CHUNK_M)],
                    gd.at[pl.ds(0, nslots * CHUNK_M)])
            hs = []
            for b in range(nslots):
                base = base_g + b * CHUNK_M
                if not first:
                    pltpu.make_async_copy(gb[b], acc.at[pl.ds(0, CHUNK_M)],
                                          ssem[b]).wait()
                hi = pltpu.async_copy(src_hbm.at[pl.ds(off + base, CHUNK_M)],
                                      ss[b], isem[b])
                if linear:
                    hg = pltpu.async_copy(nf_hbm.at[pl.ds(base, CHUNK_M)],
                                          gb[b], gsem[b])
                else:
                    hg = pltpu.async_copy(
                        nf_hbm.at[gd.at[pl.ds(b * CHUNK_M, CHUNK_M)]],
                        gb[b], gsem[b])
                he = pltpu.async_copy(
                    ef_hbm.at[pl.ds(ef_off + base, CHUNK_M)], eb[b], esem[b])
                hs.append((hi, hg, he))
            for b in range(nslots):
                hi, hg, he = hs[b]
                hg.wait()
                he.wait()
                _relu_add_rows(gb[b], eb[b], pre_relu=linear)
                hi.wait()
                pltpu.async_copy(gb[b], acc.at[ss[b]], ssem[b], add=True)

        mgroup(base_w, True)
        lax.fori_loop(
            1, ngrp,
            lambda g, carry: (mgroup(base_w + g * mgrp, False), carry)[1],
            0)
        if tail:
            mgroup(base_w + ngrp * mgrp, False, nslots=tail)
        for b in range(NBUF_M):
            pltpu.make_async_copy(gb[b], acc.at[pl.ds(0, CHUNK_M)],
                                  ssem[b]).wait()

        plsc.subcore_barrier()
        sl = pl.ds(s * ZROWS, ZROWS)
        pltpu.sync_copy(acc.at[sl], out_hbm.at[c, sl])

        @pl.when(s == NS - 1)
        def _():
            rsl = pl.ds(REM_BASE, REM)
            pltpu.sync_copy(acc.at[rsl], out_hbm.at[c, rsl])

    return pl.kernel(
        sc_message_body,
        out_type=jax.ShapeDtypeStruct((NC, N, D), jnp.float32),
        mesh=_MESH,
        scratch_types=_MSG_SCRATCH,
    )


_GATHER_SCRATCH = (
    [pltpu.VMEM((NBUF_G * CHUNK_G,), jnp.int32) for _ in range(2)]
    + [pltpu.VMEM((CHUNK_G, D), jnp.float32) for _ in range(2 * NBUF_G)]
    + [pltpu.SemaphoreType.DMA for _ in range(4 * NBUF_G)]
)


@functools.cache
def _make_sc_gather(nedges, off=0):
    epw = nedges // NW
    ggrp = NBUF_G * CHUNK_G
    ngrp = epw // ggrp

    @functools.partial(
        pl.kernel,
        out_type=(
            jax.ShapeDtypeStruct((nedges, D), jnp.float32),
            jax.ShapeDtypeStruct((nedges, D), jnp.float32),
        ),
        mesh=_MESH,
        scratch_types=_GATHER_SCRATCH,
    )
    def sc_edge_gather(nf_hbm, src_hbm, dst_hbm, gi_hbm, gj_hbm, *sc):
        isrc = sc[0]
        idst = sc[1]
        gbi = sc[2:2 + NBUF_G]
        gbj = sc[2 + NBUF_G:2 + 2 * NBUF_G]
        sems = sc[2 + 2 * NBUF_G:]
        gsi = sems[0:NBUF_G]
        gsj = sems[NBUF_G:2 * NBUF_G]
        wsi = sems[2 * NBUF_G:3 * NBUF_G]
        wsj = sems[3 * NBUF_G:4 * NBUF_G]

        c = lax.axis_index("c")
        s = lax.axis_index("s")
        wid = s * NC + c
        base_w = wid * epw

        def group(base_g, first):
            pltpu.sync_copy(src_hbm.at[pl.ds(off + base_g, ggrp)], isrc)
            pltpu.sync_copy(dst_hbm.at[pl.ds(off + base_g, ggrp)], idst)
            hs = []
            for b in range(NBUF_G):
                if not first:
                    pltpu.make_async_copy(gbi[b], gi_hbm.at[pl.ds(0, CHUNK_G)],
                                          wsi[b]).wait()
                    pltpu.make_async_copy(gbj[b], gj_hbm.at[pl.ds(0, CHUNK_G)],
                                          wsj[b]).wait()
                h1 = pltpu.async_copy(
                    nf_hbm.at[isrc.at[pl.ds(b * CHUNK_G, CHUNK_G)]],
                    gbi[b], gsi[b])
                h2 = pltpu.async_copy(
                    nf_hbm.at[idst.at[pl.ds(b * CHUNK_G, CHUNK_G)]],
                    gbj[b], gsj[b])
                hs.append((h1, h2))
            for b in range(NBUF_G):
                h1, h2 = hs[b]
                sl = pl.ds(base_g + b * CHUNK_G, CHUNK_G)
                h1.wait()
                pltpu.async_copy(gbi[b], gi_hbm.at[sl], wsi[b])
                h2.wait()
                pltpu.async_copy(gbj[b], gj_hbm.at[sl], wsj[b])

        group(base_w, True)
        lax.fori_loop(
            1, ngrp,
            lambda g, carry: (group(base_w + g * ggrp, False), carry)[1],
            0)
        for b in range(NBUF_G):
            pltpu.make_async_copy(gbi[b], gi_hbm.at[pl.ds(0, CHUNK_G)],
                                  wsi[b]).wait()
            pltpu.make_async_copy(gbj[b], gj_hbm.at[pl.ds(0, CHUNK_G)],
                                  wsj[b]).wait()

    return sc_edge_gather


def _ln(x, g, b):
    m = jnp.mean(x, axis=-1, keepdims=True)
    xc = x - m
    v = jnp.mean(xc * xc, axis=-1, keepdims=True)
    return xc * jax.lax.rsqrt(v + 1e-5) * g + b


def _ln_fast(x, g, b):
    m = jnp.mean(x, axis=-1, keepdims=True)
    m2 = jnp.mean(x * x, axis=-1, keepdims=True)
    rs = jax.lax.rsqrt(jnp.maximum(m2 - m * m, 0.0) + 1e-5)
    return (x - m) * rs * g + b


BN = 1000


def _node_mlp_body(nf, p0, p1, p2, p3, eps, w1, b1, lg1, lb1, w2, b2,
                   bng, bnb, out_ln, out_relu):
    h = ((1.0 + eps[0, 0]) * nf[...] + (p0[...] + p1[...])
         + (p2[...] + p3[...]))
    t = jnp.dot(h, w1[...], preferred_element_type=jnp.float32) + b1[...]
    t = _ln(t, lg1[...], lb1[...])
    t = jnp.maximum(t, 0.0)
    u = jnp.dot(t, w2[...], preferred_element_type=jnp.float32) + b2[...]
    y = _ln(u, bng[...], bnb[...])
    out_ln[...] = y
    out_relu[...] = jnp.maximum(y, 0.0)


def _tc_node_mlp(nf, p0, p1, p2, p3, eps, w1, b1, lg1, lb1, w2, b2, bng, bnb):
    grid = (N // BN,)
    row_spec = pl.BlockSpec((BN, D), lambda i: (i, 0))
    full = lambda shape: pl.BlockSpec(shape, lambda i: (0, 0))
    return pl.pallas_call(
        _node_mlp_body,
        grid=grid,
        in_specs=[
            row_spec, row_spec, row_spec, row_spec, row_spec,
            full((1, 1)),
            full((D, 2 * D)), full((1, 2 * D)), full((1, 2 * D)), full((1, 2 * D)),
            full((2 * D, D)), full((1, D)), full((1, D)), full((1, D)),
        ],
        out_specs=[row_spec, row_spec],
        out_shape=[
            jax.ShapeDtypeStruct((N, D), jnp.float32),
            jax.ShapeDtypeStruct((N, D), jnp.float32),
        ],
    )(nf, p0, p1, p2, p3, eps, w1, b1, lg1, lb1, w2, b2, bng, bnb)


BM = 1280


def _edge_mlp_body(gi, gj, ef, wa, wb, wc, b1, lg, lb, w2, b2, out):
    bf = jnp.bfloat16
    efv = ef[...]
    s = (gi[...] + gj[...]).astype(bf)
    a = jnp.abs(gi[...] - gj[...]).astype(bf)
    t = jnp.dot(s, wa[...], preferred_element_type=jnp.float32)
    t += jnp.dot(a, wb[...], preferred_element_type=jnp.float32)
    t += jnp.dot(efv.astype(bf), wc[...], preferred_element_type=jnp.float32)
    t += b1[...]
    t = _ln_fast(t, lg[...], lb[...])
    t = jnp.maximum(t, 0.0).astype(bf)
    u = jnp.dot(t, w2[...], preferred_element_type=jnp.float32) + b2[...]
    out[...] = u + efv


def _tc_edge_mlp(gi, gj, ef, wa, wb, wc, b1, lg, lb, w2, b2, ef_off=0):
    nedges = gi.shape[0]
    grid = (nedges // BM,)
    efb = ef_off // BM
    full = lambda shape: pl.BlockSpec(shape, lambda i: (0, 0))
    row_spec = pl.BlockSpec((BM, D), lambda i: (i, 0))
    ef_spec = pl.BlockSpec((BM, D), lambda i: (i + efb, 0))
    return pl.pallas_call(
        _edge_mlp_body,
        grid=grid,
        in_specs=[
            row_spec, row_spec, ef_spec,
            full((D, 3 * D)), full((D, 3 * D)), full((D, 3 * D)),
            full((1, 3 * D)), full((1, 3 * D)), full((1, 3 * D)),
            full((3 * D, D)), full((1, D)),
        ],
        out_specs=row_spec,
        out_shape=jax.ShapeDtypeStruct((nedges, D), jnp.float32),
    )(gi, gj, ef,
      wa.astype(jnp.bfloat16), wb.astype(jnp.bfloat16), wc.astype(jnp.bfloat16),
      b1, lg, lb, w2.astype(jnp.bfloat16), b2)


def kernel(node_feats, edge_feats, edge_index, params):
    E = edge_feats.shape[0]
    E2 = E // 2
    src = edge_index[0]
    dst = edge_index[1]
    msg0 = _make_sc_message(E2, ef_off=0)
    msg1 = _make_sc_message(E2, off=E2, ef_off=E2)
    msg_lin0 = _make_sc_message(E2, linear=True, off=0)
    msg_lin1 = _make_sc_message(E2, linear=True, off=E2)
    gat0 = _make_sc_gather(E2, off=0)
    gat1 = _make_sc_gather(E2, off=E2)
    nf = node_feats
    nf_ln = node_feats
    ef0 = ef1 = edge_feats
    gj0 = gj1 = None
    num_layers = len(params)
    for l in range(num_layers):
        p = params["layer%d" % l]
        if l == 0:
            pa = msg0(nf, ef0, src, dst)
            pb = msg1(nf, ef1, src, dst)
        else:
            pa = msg_lin0(gj0, ef0, src)
            pb = msg_lin1(gj1, ef1, src)
        nf_ln, nf_relu = _tc_node_mlp(
            nf, pa[0], pa[1], pb[0], pb[1],
            p["eps"].reshape(1, 1),
            p["cW1"], p["cb1"].reshape(1, -1),
            p["cln_g"].reshape(1, -1), p["cln_b"].reshape(1, -1),
            p["cW2"], p["cb2"].reshape(1, -1),
            p["bn_g"].reshape(1, -1), p["bn_b"].reshape(1, -1),
        )
        ew = (p["eW1"][:D], p["eW1"][D:2 * D], p["eW1"][2 * D:],
              p["eb1"].reshape(1, -1),
              p["eln_g"].reshape(1, -1), p["eln_b"].reshape(1, -1),
              p["eW2"], p["eb2"].reshape(1, -1))
        gi0, gj0 = gat0(nf_ln, src, dst)
        gi1, gj1 = gat1(nf_ln, src, dst)
        ef_off0, ef_off1 = (0, E2) if l == 0 else (0, 0)
        ef0 = _tc_edge_mlp(gi0, gj0, ef0, *ew, ef_off=ef_off0)
        ef1 = _tc_edge_mlp(gi1, gj1, ef1, *ew, ef_off=ef_off1)
        nf = nf_relu
    return nf_ln, jnp.concatenate([ef0, ef1], axis=0)


# speedup vs baseline: 1.5335x; 1.0204x over previous
import functools

import jax
import jax.numpy as jnp
from jax import lax
from jax.experimental import pallas as pl
from jax.experimental.pallas import tpu as pltpu
from jax.experimental.pallas import tpu_sc as plsc

N = 10000
D = 128
NC = 2
NS = 16
NW = NC * NS
ZROWS = 624
REM_BASE = NS * ZROWS
REM = N - REM_BASE
ZBUF = 16

CHUNK_G = 40
NBUF_G = 5

CHUNK_M = 40
NBUF_M = 4

_MESH = plsc.VectorSubcoreMesh(
    core_axis_name="c", subcore_axis_name="s", num_cores=NC, num_subcores=NS)


def _relu_add_rows(g_ref, e_ref, pre_relu=False):
    def row(r, carry):
        for k in range(D // 16):
            sl = pl.ds(k * 16, 16)
            g = g_ref[r, sl]
            if pre_relu:
                g = jnp.maximum(g, 0.0)
            g_ref[r, sl] = jnp.maximum(g + e_ref[r, sl], 0.0)
        return carry
    lax.fori_loop(0, CHUNK_M, row, 0, unroll=2)


_MSG_SCRATCH = (
    [pltpu.VMEM_SHARED((N, D), jnp.float32)]
    + [pltpu.VMEM((NBUF_M * CHUNK_M,), jnp.int32)]
    + [pltpu.VMEM((CHUNK_M,), jnp.int32) for _ in range(NBUF_M)]
    + [pltpu.VMEM((CHUNK_M, D), jnp.float32) for _ in range(NBUF_M)]
    + [pltpu.VMEM((CHUNK_M, D), jnp.float32) for _ in range(NBUF_M)]
    + [pltpu.VMEM((ZBUF, D), jnp.float32)]
    + [pltpu.SemaphoreType.DMA for _ in range(4 * NBUF_M)]
)


@functools.cache
def _make_sc_message(nedges, linear=False, off=0, ef_off=0):
    epw = nedges // NW
    nchunk = epw // CHUNK_M
    mgrp = NBUF_M * CHUNK_M
    ngrp = nchunk // NBUF_M
    tail = nchunk - ngrp * NBUF_M

    def sc_message_body(*refs):
        if linear:
            nf_hbm, ef_hbm, src_hbm, out_hbm = refs[:4]
            dst_hbm = None
            sc = refs[4:]
        else:
            nf_hbm, ef_hbm, src_hbm, dst_hbm, out_hbm = refs[:5]
            sc = refs[5:]
        acc = sc[0]
        gd = sc[1]
        ss = sc[2:2 + NBUF_M]
        gb = sc[2 + NBUF_M:2 + 2 * NBUF_M]
        eb = sc[2 + 2 * NBUF_M:2 + 3 * NBUF_M]
        zbuf = sc[2 + 3 * NBUF_M]
        sems = sc[3 + 3 * NBUF_M:]
        isem = sems[0:NBUF_M]
        gsem = sems[NBUF_M:2 * NBUF_M]
        esem = sems[2 * NBUF_M:3 * NBUF_M]
        ssem = sems[3 * NBUF_M:4 * NBUF_M]

        c = lax.axis_index("c")
        s = lax.axis_index("s")
        wid = s * NC + c
        base_w = wid * epw

        def zrow(r, carry):
            for k in range(D // 16):
                zbuf[r, pl.ds(k * 16, 16)] = jnp.zeros((16,), jnp.float32)
            return carry
        lax.fori_loop(0, ZBUF, zrow, 0)
        for j in range(ZROWS // ZBUF):
            pltpu.sync_copy(zbuf, acc.at[pl.ds(s * ZROWS + j * ZBUF, ZBUF)])

        @pl.when(s == NS - 1)
        def _():
            pltpu.sync_copy(zbuf.at[pl.ds(0, REM)],
                            acc.at[pl.ds(REM_BASE, REM)])
        plsc.subcore_barrier()

        def mgroup(base_g, first, nslots=NBUF_M):
            if not linear:
                pltpu.sync_copy(
                    dst_hbm.at[pl.ds(off + base_g, nslots * CHUNK_M)],
                    gd.at[pl.ds(0, nslots * CHUNK_M)])
            hs = []
            for b in range(nslots):
                base = base_g + b * CHUNK_M
                if not first:
                    pltpu.make_async_copy(gb[b], acc.at[pl.ds(0, CHUNK_M)],
                                          ssem[b]).wait()
                hi = pltpu.async_copy(src_hbm.at[pl.ds(off + base, CHUNK_M)],
                                      ss[b], isem[b])
                if linear:
                    hg = pltpu.async_copy(nf_hbm.at[pl.ds(base, CHUNK_M)],
                                          gb[b], gsem[b])
                else:
                    hg = pltpu.async_copy(
                        nf_hbm.at[gd.at[pl.ds(b * CHUNK_M, CHUNK_M)]],
                        gb[b], gsem[b])
                he = pltpu.async_copy(
                    ef_hbm.at[pl.ds(ef_off + base, CHUNK_M)], eb[b], esem[b])
                hs.append((hi, hg, he))
            for b in range(nslots):
                hi, hg, he = hs[b]
                hg.wait()
                he.wait()
                _relu_add_rows(gb[b], eb[b], pre_relu=linear)
                hi.wait()
                pltpu.async_copy(gb[b], acc.at[ss[b]], ssem[b], add=True)

        mgroup(base_w, True)
        lax.fori_loop(
            1, ngrp,
            lambda g, carry: (mgroup(base_w + g * mgrp, False), carry)[1],
            0)
        if tail:
            mgroup(base_w + ngrp * mgrp, False, nslots=tail)
        for b in range(NBUF_M):
            pltpu.make_async_copy(gb[b], acc.at[pl.ds(0, CHUNK_M)],
                                  ssem[b]).wait()

        plsc.subcore_barrier()
        sl = pl.ds(s * ZROWS, ZROWS)
        pltpu.sync_copy(acc.at[sl], out_hbm.at[c, sl])

        @pl.when(s == NS - 1)
        def _():
            rsl = pl.ds(REM_BASE, REM)
            pltpu.sync_copy(acc.at[rsl], out_hbm.at[c, rsl])

    return pl.kernel(
        sc_message_body,
        out_type=jax.ShapeDtypeStruct((NC, N, D), jnp.float32),
        mesh=_MESH,
        scratch_types=_MSG_SCRATCH,
    )


_GATHER_SCRATCH = (
    [pltpu.VMEM((NBUF_G * CHUNK_G,), jnp.int32) for _ in range(2)]
    + [pltpu.VMEM((CHUNK_G, D), jnp.float32) for _ in range(2 * NBUF_G)]
    + [pltpu.SemaphoreType.DMA for _ in range(4 * NBUF_G)]
)


@functools.cache
def _make_sc_gather(nedges, off=0):
    epw = nedges // NW
    ggrp = NBUF_G * CHUNK_G
    ngrp = epw // ggrp

    @functools.partial(
        pl.kernel,
        out_type=(
            jax.ShapeDtypeStruct((nedges, D), jnp.float32),
            jax.ShapeDtypeStruct((nedges, D), jnp.float32),
        ),
        mesh=_MESH,
        scratch_types=_GATHER_SCRATCH,
    )
    def sc_edge_gather(nf_hbm, src_hbm, dst_hbm, gi_hbm, gj_hbm, *sc):
        isrc = sc[0]
        idst = sc[1]
        gbi = sc[2:2 + NBUF_G]
        gbj = sc[2 + NBUF_G:2 + 2 * NBUF_G]
        sems = sc[2 + 2 * NBUF_G:]
        gsi = sems[0:NBUF_G]
        gsj = sems[NBUF_G:2 * NBUF_G]
        wsi = sems[2 * NBUF_G:3 * NBUF_G]
        wsj = sems[3 * NBUF_G:4 * NBUF_G]

        c = lax.axis_index("c")
        s = lax.axis_index("s")
        wid = s * NC + c
        base_w = wid * epw

        def group(base_g, first):
            pltpu.sync_copy(src_hbm.at[pl.ds(off + base_g, ggrp)], isrc)
            pltpu.sync_copy(dst_hbm.at[pl.ds(off + base_g, ggrp)], idst)
            hs = []
            for b in range(NBUF_G):
                if not first:
                    pltpu.make_async_copy(gbi[b], gi_hbm.at[pl.ds(0, CHUNK_G)],
                                          wsi[b]).wait()
                    pltpu.make_async_copy(gbj[b], gj_hbm.at[pl.ds(0, CHUNK_G)],
                                          wsj[b]).wait()
                h1 = pltpu.async_copy(
                    nf_hbm.at[isrc.at[pl.ds(b * CHUNK_G, CHUNK_G)]],
                    gbi[b], gsi[b])
                h2 = pltpu.async_copy(
                    nf_hbm.at[idst.at[pl.ds(b * CHUNK_G, CHUNK_G)]],
                    gbj[b], gsj[b])
                hs.append((h1, h2))
            for b in range(NBUF_G):
                h1, h2 = hs[b]
                sl = pl.ds(base_g + b * CHUNK_G, CHUNK_G)
                h1.wait()
                pltpu.async_copy(gbi[b], gi_hbm.at[sl], wsi[b])
                h2.wait()
                pltpu.async_copy(gbj[b], gj_hbm.at[sl], wsj[b])

        group(base_w, True)
        lax.fori_loop(
            1, ngrp,
            lambda g, carry: (group(base_w + g * ggrp, False), carry)[1],
            0)
        for b in range(NBUF_G):
            pltpu.make_async_copy(gbi[b], gi_hbm.at[pl.ds(0, CHUNK_G)],
                                  wsi[b]).wait()
            pltpu.make_async_copy(gbj[b], gj_hbm.at[pl.ds(0, CHUNK_G)],
                                  wsj[b]).wait()

    return sc_edge_gather


def _ln(x, g, b):
    m = jnp.mean(x, axis=-1, keepdims=True)
    xc = x - m
    v = jnp.mean(xc * xc, axis=-1, keepdims=True)
    return xc * jax.lax.rsqrt(v + 1e-5) * g + b


def _ln_fast(x, g, b):
    m = jnp.mean(x, axis=-1, keepdims=True)
    m2 = jnp.mean(x * x, axis=-1, keepdims=True)
    rs = jax.lax.rsqrt(jnp.maximum(m2 - m * m, 0.0) + 1e-5)
    return (x - m) * rs * g + b


BN = 1000


def _node_mlp_body(nf, p0, p1, p2, p3, eps, w1, b1, lg1, lb1, w2, b2,
                   bng, bnb, out_ln, out_relu):
    h = ((1.0 + eps[0, 0]) * nf[...] + (p0[...] + p1[...])
         + (p2[...] + p3[...]))
    t = jnp.dot(h, w1[...], preferred_element_type=jnp.float32) + b1[...]
    t = _ln(t, lg1[...], lb1[...])
    t = jnp.maximum(t, 0.0)
    u = jnp.dot(t, w2[...], preferred_element_type=jnp.float32) + b2[...]
    y = _ln(u, bng[...], bnb[...])
    out_ln[...] = y
    out_relu[...] = jnp.maximum(y, 0.0)


def _tc_node_mlp(nf, p0, p1, p2, p3, eps, w1, b1, lg1, lb1, w2, b2, bng, bnb):
    grid = (N // BN,)
    row_spec = pl.BlockSpec((BN, D), lambda i: (i, 0))
    full = lambda shape: pl.BlockSpec(shape, lambda i: (0, 0))
    return pl.pallas_call(
        _node_mlp_body,
        grid=grid,
        in_specs=[
            row_spec, row_spec, row_spec, row_spec, row_spec,
            full((1, 1)),
            full((D, 2 * D)), full((1, 2 * D)), full((1, 2 * D)), full((1, 2 * D)),
            full((2 * D, D)), full((1, D)), full((1, D)), full((1, D)),
        ],
        out_specs=[row_spec, row_spec],
        out_shape=[
            jax.ShapeDtypeStruct((N, D), jnp.float32),
            jax.ShapeDtypeStruct((N, D), jnp.float32),
        ],
    )(nf, p0, p1, p2, p3, eps, w1, b1, lg1, lb1, w2, b2, bng, bnb)


BM = 1600


def _edge_mlp_body(gi, gj, ef, wa, wb, wc, b1, lg, lb, w2, b2, out):
    bf = jnp.bfloat16
    efv = ef[...]
    s = (gi[...] + gj[...]).astype(bf)
    a = jnp.abs(gi[...] - gj[...]).astype(bf)
    t = jnp.dot(s, wa[...], preferred_element_type=jnp.float32)
    t += jnp.dot(a, wb[...], preferred_element_type=jnp.float32)
    t += jnp.dot(efv.astype(bf), wc[...], preferred_element_type=jnp.float32)
    t += b1[...]
    m = jnp.mean(t, axis=-1, keepdims=True)
    m2 = jnp.mean(t * t, axis=-1, keepdims=True)
    rs = jax.lax.rsqrt(jnp.maximum(m2 - m * m, 0.0) + 1e-5)
    t16 = (t - m).astype(bf) * rs.astype(bf) * lg[...] + lb[...]
    t16 = jnp.maximum(t16, jnp.array(0.0, bf))
    u = jnp.dot(t16, w2[...], preferred_element_type=jnp.float32) + b2[...]
    out[...] = u + efv


def _tc_edge_mlp(gi, gj, ef, wa, wb, wc, b1, lg, lb, w2, b2, ef_off=0):
    nedges = gi.shape[0]
    grid = (nedges // BM,)
    efb = ef_off // BM
    full = lambda shape: pl.BlockSpec(shape, lambda i: (0, 0))
    row_spec = pl.BlockSpec((BM, D), lambda i: (i, 0))
    ef_spec = pl.BlockSpec((BM, D), lambda i: (i + efb, 0))
    return pl.pallas_call(
        _edge_mlp_body,
        grid=grid,
        in_specs=[
            row_spec, row_spec, ef_spec,
            full((D, 3 * D)), full((D, 3 * D)), full((D, 3 * D)),
            full((1, 3 * D)), full((1, 3 * D)), full((1, 3 * D)),
            full((3 * D, D)), full((1, D)),
        ],
        out_specs=row_spec,
        out_shape=jax.ShapeDtypeStruct((nedges, D), jnp.float32),
    )(gi, gj, ef,
      wa.astype(jnp.bfloat16), wb.astype(jnp.bfloat16), wc.astype(jnp.bfloat16),
      b1, lg.astype(jnp.bfloat16), lb.astype(jnp.bfloat16),
      w2.astype(jnp.bfloat16), b2)


def kernel(node_feats, edge_feats, edge_index, params):
    E = edge_feats.shape[0]
    E2 = E // 2
    src = edge_index[0]
    dst = edge_index[1]
    msg0 = _make_sc_message(E2, ef_off=0)
    msg1 = _make_sc_message(E2, off=E2, ef_off=E2)
    msg_lin0 = _make_sc_message(E2, linear=True, off=0)
    msg_lin1 = _make_sc_message(E2, linear=True, off=E2)
    gat0 = _make_sc_gather(E2, off=0)
    gat1 = _make_sc_gather(E2, off=E2)
    nf = node_feats
    nf_ln = node_feats
    ef0 = ef1 = edge_feats
    gj0 = gj1 = None
    num_layers = len(params)
    for l in range(num_layers):
        p = params["layer%d" % l]
        if l == 0:
            pa = msg0(nf, ef0, src, dst)
            pb = msg1(nf, ef1, src, dst)
        else:
            pa = msg_lin0(gj0, ef0, src)
            pb = msg_lin1(gj1, ef1, src)
        nf_ln, nf_relu = _tc_node_mlp(
            nf, pa[0], pa[1], pb[0], pb[1],
            p["eps"].reshape(1, 1),
            p["cW1"], p["cb1"].reshape(1, -1),
            p["cln_g"].reshape(1, -1), p["cln_b"].reshape(1, -1),
            p["cW2"], p["cb2"].reshape(1, -1),
            p["bn_g"].reshape(1, -1), p["bn_b"].reshape(1, -1),
        )
        ew = (p["eW1"][:D], p["eW1"][D:2 * D], p["eW1"][2 * D:],
              p["eb1"].reshape(1, -1),
              p["eln_g"].reshape(1, -1), p["eln_b"].reshape(1, -1),
              p["eW2"], p["eb2"].reshape(1, -1))
        gi0, gj0 = gat0(nf_ln, src, dst)
        gi1, gj1 = gat1(nf_ln, src, dst)
        ef_off0, ef_off1 = (0, E2) if l == 0 else (0, 0)
        ef0 = _tc_edge_mlp(gi0, gj0, ef0, *ew, ef_off=ef_off0)
        ef1 = _tc_edge_mlp(gi1, gj1, ef1, *ew, ef_off=ef_off1)
        nf = nf_relu
    return nf_ln, jnp.concatenate([ef0, ef1], axis=0)
